# Initial kernel scaffold; baseline (speedup 1.0000x reference)
import jax
import jax.numpy as jnp
from jax.experimental import pallas as pl


def kernel(x, edge_index, edge_label_index, edge_type, convN1_w, convN1_b, convE1_w, convE1_b, convN2_w, convN2_b, convE2_w, convE2_b, cls_w, cls_b):
    raise NotImplementedError("write your pallas kernel here")


# speedup vs baseline: 1.0026x; 1.0026x over previous
import functools

import jax
import jax.numpy as jnp
from jax.experimental import pallas as pl
from jax.experimental.pallas import tpu as pltpu

_HIDDEN = 32
_SIMILAR_WEIGHT = 1.0
_REGULARIZER = 0.1


def _round_up(n, m):
    return ((n + m - 1) // m) * m


def _pick_tile(padded, want):
    return want if padded % want == 0 else padded


def _pad_rows(a, rows):
    pad = rows - a.shape[0]
    if pad == 0:
        return a
    return jnp.pad(a, ((0, pad), (0, 0)))


def _agg1_kernel(dst_ref, ea_ref, we_ref, b_ref, out_ref, acc_ref, deg_ref,
                 *, tn):
    i = pl.program_id(0)
    k = pl.program_id(1)

    @pl.when(k == 0)
    def _():
        acc_ref[...] = jnp.zeros_like(acc_ref)
        deg_ref[...] = jnp.zeros_like(deg_ref)

    msg = jnp.dot(ea_ref[...], we_ref[...], preferred_element_type=jnp.float32)
    msg = jnp.maximum(msg + b_ref[...], 0.0)
    te = msg.shape[0]
    ids = i * tn + jax.lax.broadcasted_iota(jnp.int32, (tn, te), 0)
    onehot = (dst_ref[...] == ids).astype(jnp.float32)
    acc_ref[...] += jnp.dot(onehot, msg, preferred_element_type=jnp.float32)
    deg_ref[...] += jnp.sum(onehot, axis=1, keepdims=True)

    @pl.when(k == pl.num_programs(1) - 1)
    def _():
        out_ref[...] = acc_ref[...] / jnp.maximum(deg_ref[...], 1.0)


def _agg1(dst_row, ea, we, b, *, n_pad, tn, e_pad, te):
    h = we.shape[1]
    fe = ea.shape[1]
    return pl.pallas_call(
        functools.partial(_agg1_kernel, tn=tn),
        out_shape=jax.ShapeDtypeStruct((n_pad, h), jnp.float32),
        grid=(n_pad // tn, e_pad // te),
        in_specs=[
            pl.BlockSpec((1, te), lambda i, k: (0, k)),
            pl.BlockSpec((te, fe), lambda i, k: (k, 0)),
            pl.BlockSpec((fe, h), lambda i, k: (0, 0)),
            pl.BlockSpec((1, h), lambda i, k: (0, 0)),
        ],
        out_specs=pl.BlockSpec((tn, h), lambda i, k: (i, 0)),
        scratch_shapes=[pltpu.VMEM((tn, h), jnp.float32),
                        pltpu.VMEM((tn, 1), jnp.float32)],
        compiler_params=pltpu.CompilerParams(
            dimension_semantics=("parallel", "arbitrary")),
    )(dst_row, ea, we, b.reshape(1, h))


def _edge_kernel(xs_ref, xd_ref, ea_ref, w1_ref, w2_ref, w3_ref, b_ref,
                 wx2_ref, we2_ref, b2_ref, w3h_ref, m2_ref, p_ref):
    ea1 = jnp.dot(xs_ref[...], w1_ref[...], preferred_element_type=jnp.float32)
    ea1 = ea1 + jnp.dot(xd_ref[...], w2_ref[...],
                        preferred_element_type=jnp.float32)
    ea1 = ea1 + jnp.dot(ea_ref[...], w3_ref[...],
                        preferred_element_type=jnp.float32)
    ea1 = jnp.maximum(ea1 + b_ref[...], 0.0)
    m2 = jnp.dot(xs_ref[...], wx2_ref[...], preferred_element_type=jnp.float32)
    m2 = m2 + jnp.dot(ea1, we2_ref[...], preferred_element_type=jnp.float32)
    m2_ref[...] = jnp.maximum(m2 + b2_ref[...], 0.0)
    p_ref[...] = jnp.dot(ea1, w3h_ref[...], preferred_element_type=jnp.float32)


def _edge_pass(xs, xd, ea, w1, w2, w3, b, wx2, we2, b2, w3h, *, e_pad, te):
    h = w1.shape[1]
    fn = xs.shape[1]
    fe = ea.shape[1]
    return pl.pallas_call(
        _edge_kernel,
        out_shape=(jax.ShapeDtypeStruct((e_pad, h), jnp.float32),
                   jax.ShapeDtypeStruct((e_pad, h), jnp.float32)),
        grid=(e_pad // te,),
        in_specs=[
            pl.BlockSpec((te, fn), lambda i: (i, 0)),
            pl.BlockSpec((te, fn), lambda i: (i, 0)),
            pl.BlockSpec((te, fe), lambda i: (i, 0)),
            pl.BlockSpec((fn, h), lambda i: (0, 0)),
            pl.BlockSpec((fn, h), lambda i: (0, 0)),
            pl.BlockSpec((fe, h), lambda i: (0, 0)),
            pl.BlockSpec((1, h), lambda i: (0, 0)),
            pl.BlockSpec((h, h), lambda i: (0, 0)),
            pl.BlockSpec((h, h), lambda i: (0, 0)),
            pl.BlockSpec((1, h), lambda i: (0, 0)),
            pl.BlockSpec((h, h), lambda i: (0, 0)),
        ],
        out_specs=(pl.BlockSpec((te, h), lambda i: (i, 0)),
                   pl.BlockSpec((te, h), lambda i: (i, 0))),
        compiler_params=pltpu.CompilerParams(dimension_semantics=("parallel",)),
    )(xs, xd, ea, w1, w2, w3, b.reshape(1, h), wx2, we2, b2.reshape(1, h), w3h)


def _agg2_kernel(dst_ref, m2_ref, w1h_ref, w2h_ref, bh_ref,
                 a_ref, b_out_ref, acc_ref, deg_ref, *, tn):
    i = pl.program_id(0)
    k = pl.program_id(1)

    @pl.when(k == 0)
    def _():
        acc_ref[...] = jnp.zeros_like(acc_ref)
        deg_ref[...] = jnp.zeros_like(deg_ref)

    m2 = m2_ref[...]
    te = m2.shape[0]
    ids = i * tn + jax.lax.broadcasted_iota(jnp.int32, (tn, te), 0)
    onehot = (dst_ref[...] == ids).astype(jnp.float32)
    acc_ref[...] += jnp.dot(onehot, m2, preferred_element_type=jnp.float32)
    deg_ref[...] += jnp.sum(onehot, axis=1, keepdims=True)

    @pl.when(k == pl.num_programs(1) - 1)
    def _():
        x2 = acc_ref[...] / jnp.maximum(deg_ref[...], 1.0)
        a_ref[...] = jnp.dot(x2, w1h_ref[...],
                             preferred_element_type=jnp.float32) + bh_ref[...]
        b_out_ref[...] = jnp.dot(x2, w2h_ref[...],
                                 preferred_element_type=jnp.float32)


def _agg2(dst_row, m2, w1h, w2h, bh, *, n_pad, tn, e_pad, te):
    h = w1h.shape[1]
    return pl.pallas_call(
        functools.partial(_agg2_kernel, tn=tn),
        out_shape=(jax.ShapeDtypeStruct((n_pad, h), jnp.float32),
                   jax.ShapeDtypeStruct((n_pad, h), jnp.float32)),
        grid=(n_pad // tn, e_pad // te),
        in_specs=[
            pl.BlockSpec((1, te), lambda i, k: (0, k)),
            pl.BlockSpec((te, h), lambda i, k: (k, 0)),
            pl.BlockSpec((h, h), lambda i, k: (0, 0)),
            pl.BlockSpec((h, h), lambda i, k: (0, 0)),
            pl.BlockSpec((1, h), lambda i, k: (0, 0)),
        ],
        out_specs=(pl.BlockSpec((tn, h), lambda i, k: (i, 0)),
                   pl.BlockSpec((tn, h), lambda i, k: (i, 0))),
        scratch_shapes=[pltpu.VMEM((tn, h), jnp.float32),
                        pltpu.VMEM((tn, 1), jnp.float32)],
        compiler_params=pltpu.CompilerParams(
            dimension_semantics=("parallel", "arbitrary")),
    )(dst_row, m2, w1h, w2h, bh.reshape(1, h))


def _head_kernel(g_ref, s_ref, y_ref, bc_ref, pred_ref, part_ref, *, sw):
    hid = jnp.maximum(g_ref[...], 0.0)
    logits = jnp.dot(hid, s_ref[...], preferred_element_type=jnp.float32)
    logits = logits + bc_ref[...]
    p = jax.nn.sigmoid(logits)
    pred_ref[...] = p

    y = y_ref[...]
    w = jnp.where(p > 0.5, jnp.float32(sw), jnp.float32(1.0))
    log_p = jnp.maximum(jnp.log(p), -100.0)
    log_1mp = jnp.maximum(jnp.log(1.0 - p), -100.0)
    bce = w * -(y * log_p + (1.0 - y) * log_1mp)

    p2 = p * p
    rows = jnp.concatenate([
        jnp.sum(p, axis=0, keepdims=True),
        jnp.sum(p2, axis=0, keepdims=True),
        jnp.sum(p2 * p, axis=0, keepdims=True),
        jnp.sum(p2 * p2, axis=0, keepdims=True),
        jnp.sum(bce, axis=0, keepdims=True),
        jnp.zeros((3, 128), jnp.float32),
    ], axis=0)
    part_ref[...] = rows.reshape(1, 8, 128)


def _head(g2d, s_mat, y2d, bc, *, rows, tl, sw):
    n_tiles = rows // tl
    lanes = g2d.shape[1]
    return pl.pallas_call(
        functools.partial(_head_kernel, sw=sw),
        out_shape=(jax.ShapeDtypeStruct((rows, 128), jnp.float32),
                   jax.ShapeDtypeStruct((n_tiles, 8, 128), jnp.float32)),
        grid=(n_tiles,),
        in_specs=[
            pl.BlockSpec((tl, lanes), lambda i: (i, 0)),
            pl.BlockSpec((lanes, 128), lambda i: (0, 0)),
            pl.BlockSpec((tl, 128), lambda i: (i, 0)),
            pl.BlockSpec((1, 1), lambda i: (0, 0)),
        ],
        out_specs=(pl.BlockSpec((tl, 128), lambda i: (i, 0)),
                   pl.BlockSpec((1, 8, 128), lambda i: (i, 0, 0))),
        compiler_params=pltpu.CompilerParams(dimension_semantics=("parallel",)),
    )(g2d, s_mat, y2d, bc)


def _loss_kernel(part_ref, loss_ref, *, n, reg):
    v = jnp.sum(part_ref[...], axis=0)
    s = jnp.sum(v, axis=1, keepdims=True)
    sp, sp2, sp3, sp4, sbce = (s[0:1, :], s[1:2, :], s[2:3, :], s[3:4, :],
                               s[4:5, :])
    nf = jnp.float32(n)
    mu = sp / nf
    mu2 = mu * mu
    m4 = (sp4 / nf - 4.0 * mu * (sp3 / nf)
          + 6.0 * mu2 * (sp2 / nf) - 3.0 * mu2 * mu2)
    m4 = jnp.maximum(m4, 0.0)
    loss_ref[...] = sbce / nf - jnp.sqrt(jnp.sqrt(m4)) * reg


def _finalize(part, *, n, reg):
    n_tiles = part.shape[0]
    return pl.pallas_call(
        functools.partial(_loss_kernel, n=n, reg=reg),
        out_shape=jax.ShapeDtypeStruct((1, 1), jnp.float32),
        grid=(1,),
        in_specs=[pl.BlockSpec((n_tiles, 8, 128), lambda i: (0, 0, 0))],
        out_specs=pl.BlockSpec((1, 1), lambda i: (0, 0)),
    )(part)


def kernel(x, edge_index, edge_label_index, edge_type,
           convN1_w, convN1_b, convE1_w, convE1_b,
           convN2_w, convN2_b, convE2_w, convE2_b, cls_w, cls_b):
    x = x.astype(jnp.float32)
    n_nodes, f_in = x.shape
    n_edges = edge_index.shape[1]
    n_labels = edge_label_index.shape[1]
    h = _HIDDEN

    src, dst = edge_index[0], edge_index[1]
    lsrc, ldst = edge_label_index[0], edge_label_index[1]

    ea0 = (x[dst] - x[src]).astype(jnp.float32)

    n_pad = _round_up(n_nodes, 8)
    tn = _pick_tile(n_pad, 512)
    e_pad = _round_up(n_edges, 128)
    te = _pick_tile(e_pad, 1024)

    dst_row = jnp.full((1, e_pad), -1, jnp.int32).at[0, :n_edges].set(
        dst.astype(jnp.int32))
    ea0p = _pad_rows(ea0, e_pad)

    we1 = convN1_w[f_in:2 * f_in]
    X1 = _agg1(dst_row, ea0p, we1, convN1_b,
               n_pad=n_pad, tn=tn, e_pad=e_pad, te=te)[:n_nodes]

    xs1 = _pad_rows(X1[src], e_pad)
    xd1 = _pad_rows(X1[dst], e_pad)

    w1, w2, w3 = convE1_w[:h], convE1_w[h:2 * h], convE1_w[2 * h:2 * h + f_in]
    wx2, we2 = convN2_w[:h], convN2_w[h:2 * h]
    w1h, w2h, w3h = convE2_w[:h], convE2_w[h:2 * h], convE2_w[2 * h:3 * h]
    te_b = _pick_tile(e_pad, 2048)
    M2, P = _edge_pass(xs1, xd1, ea0p, w1, w2, w3, convE1_b,
                       wx2, we2, convN2_b, w3h, e_pad=e_pad, te=te_b)

    a_tab, b_tab = _agg2(dst_row, M2, w1h, w2h, convE2_b,
                         n_pad=n_pad, tn=tn, e_pad=e_pad, te=te)
    a_tab, b_tab = a_tab[:n_nodes], b_tab[:n_nodes]

    key = src.astype(jnp.int32) * n_nodes + dst.astype(jnp.int32)
    order = jnp.argsort(key)
    skey = key[order]
    ps = P[:n_edges][order]
    csum = jnp.concatenate(
        [jnp.zeros((1, h), jnp.float32), jnp.cumsum(ps, axis=0)], axis=0)
    qkey = lsrc.astype(jnp.int32) * n_nodes + ldst.astype(jnp.int32)
    hi = jnp.searchsorted(skey, qkey, side="right")
    lo = jnp.searchsorted(skey, qkey, side="left")

    g = a_tab[lsrc] + b_tab[ldst] + (csum[hi] - csum[lo])
    rows = n_labels // 128
    g2d = g.reshape(rows, 128 * h)
    y2d = edge_type.astype(jnp.float32).reshape(rows, 128)

    wc = cls_w.reshape(h)
    lane_ids = jnp.arange(128 * h, dtype=jnp.int32)
    s_mat = (jnp.tile(wc, 128)[:, None]
             * (lane_ids[:, None] // h
                == jnp.arange(128, dtype=jnp.int32)[None, :]
                ).astype(jnp.float32))

    tl = _pick_tile(rows, 256)
    pred2d, part = _head(g2d, s_mat, y2d, cls_b.reshape(1, 1),
                         rows=rows, tl=tl, sw=float(_SIMILAR_WEIGHT))

    loss = _finalize(part, n=n_labels, reg=float(_REGULARIZER))
    return pred2d.reshape(n_labels), loss[0, 0]


# speedup vs baseline: 17.3520x; 17.3077x over previous
import functools

import jax
import jax.numpy as jnp
from jax.experimental import pallas as pl
from jax.experimental.pallas import tpu as pltpu

_HIDDEN = 32
_SIMILAR_WEIGHT = 1.0
_REGULARIZER = 0.1


def _round_up(n, m):
    return ((n + m - 1) // m) * m


def _pick_tile(padded, want):
    return want if padded % want == 0 else padded


def _pad_rows(a, rows):
    pad = rows - a.shape[0]
    if pad == 0:
        return a
    return jnp.pad(a, ((0, pad), (0, 0)))


def _agg1_kernel(dst_ref, ea_ref, we_ref, b_ref, out_ref, acc_ref, deg_ref,
                 *, tn):
    i = pl.program_id(0)
    k = pl.program_id(1)

    @pl.when(k == 0)
    def _():
        acc_ref[...] = jnp.zeros_like(acc_ref)
        deg_ref[...] = jnp.zeros_like(deg_ref)

    msg = jnp.dot(ea_ref[...], we_ref[...], preferred_element_type=jnp.float32)
    msg = jnp.maximum(msg + b_ref[...], 0.0)
    te = msg.shape[0]
    ids = i * tn + jax.lax.broadcasted_iota(jnp.int32, (tn, te), 0)
    onehot = (dst_ref[...] == ids).astype(jnp.float32)
    acc_ref[...] += jnp.dot(onehot, msg, preferred_element_type=jnp.float32)
    deg_ref[...] += jnp.sum(onehot, axis=1, keepdims=True)

    @pl.when(k == pl.num_programs(1) - 1)
    def _():
        out_ref[...] = acc_ref[...] / jnp.maximum(deg_ref[...], 1.0)


def _agg1(dst_row, ea, we, b, *, n_pad, tn, e_pad, te):
    h = we.shape[1]
    fe = ea.shape[1]
    return pl.pallas_call(
        functools.partial(_agg1_kernel, tn=tn),
        out_shape=jax.ShapeDtypeStruct((n_pad, h), jnp.float32),
        grid=(n_pad // tn, e_pad // te),
        in_specs=[
            pl.BlockSpec((1, te), lambda i, k: (0, k)),
            pl.BlockSpec((te, fe), lambda i, k: (k, 0)),
            pl.BlockSpec((fe, h), lambda i, k: (0, 0)),
            pl.BlockSpec((1, h), lambda i, k: (0, 0)),
        ],
        out_specs=pl.BlockSpec((tn, h), lambda i, k: (i, 0)),
        scratch_shapes=[pltpu.VMEM((tn, h), jnp.float32),
                        pltpu.VMEM((tn, 1), jnp.float32)],
        compiler_params=pltpu.CompilerParams(
            dimension_semantics=("parallel", "arbitrary")),
    )(dst_row, ea, we, b.reshape(1, h))


def _edge_kernel(xs_ref, xd_ref, ea_ref, w1_ref, w2_ref, w3_ref, b_ref,
                 wx2_ref, we2_ref, b2_ref, w3h_ref, m2_ref, p_ref):
    ea1 = jnp.dot(xs_ref[...], w1_ref[...], preferred_element_type=jnp.float32)
    ea1 = ea1 + jnp.dot(xd_ref[...], w2_ref[...],
                        preferred_element_type=jnp.float32)
    ea1 = ea1 + jnp.dot(ea_ref[...], w3_ref[...],
                        preferred_element_type=jnp.float32)
    ea1 = jnp.maximum(ea1 + b_ref[...], 0.0)
    m2 = jnp.dot(xs_ref[...], wx2_ref[...], preferred_element_type=jnp.float32)
    m2 = m2 + jnp.dot(ea1, we2_ref[...], preferred_element_type=jnp.float32)
    m2_ref[...] = jnp.maximum(m2 + b2_ref[...], 0.0)
    p_ref[...] = jnp.dot(ea1, w3h_ref[...], preferred_element_type=jnp.float32)


def _edge_pass(xs, xd, ea, w1, w2, w3, b, wx2, we2, b2, w3h, *, e_pad, te):
    h = w1.shape[1]
    fn = xs.shape[1]
    fe = ea.shape[1]
    return pl.pallas_call(
        _edge_kernel,
        out_shape=(jax.ShapeDtypeStruct((e_pad, h), jnp.float32),
                   jax.ShapeDtypeStruct((e_pad, h), jnp.float32)),
        grid=(e_pad // te,),
        in_specs=[
            pl.BlockSpec((te, fn), lambda i: (i, 0)),
            pl.BlockSpec((te, fn), lambda i: (i, 0)),
            pl.BlockSpec((te, fe), lambda i: (i, 0)),
            pl.BlockSpec((fn, h), lambda i: (0, 0)),
            pl.BlockSpec((fn, h), lambda i: (0, 0)),
            pl.BlockSpec((fe, h), lambda i: (0, 0)),
            pl.BlockSpec((1, h), lambda i: (0, 0)),
            pl.BlockSpec((h, h), lambda i: (0, 0)),
            pl.BlockSpec((h, h), lambda i: (0, 0)),
            pl.BlockSpec((1, h), lambda i: (0, 0)),
            pl.BlockSpec((h, h), lambda i: (0, 0)),
        ],
        out_specs=(pl.BlockSpec((te, h), lambda i: (i, 0)),
                   pl.BlockSpec((te, h), lambda i: (i, 0))),
        compiler_params=pltpu.CompilerParams(dimension_semantics=("parallel",)),
    )(xs, xd, ea, w1, w2, w3, b.reshape(1, h), wx2, we2, b2.reshape(1, h), w3h)


def _agg2_kernel(dst_ref, m2_ref, w1h_ref, w2h_ref, bh_ref,
                 a_ref, b_out_ref, acc_ref, deg_ref, *, tn):
    i = pl.program_id(0)
    k = pl.program_id(1)

    @pl.when(k == 0)
    def _():
        acc_ref[...] = jnp.zeros_like(acc_ref)
        deg_ref[...] = jnp.zeros_like(deg_ref)

    m2 = m2_ref[...]
    te = m2.shape[0]
    ids = i * tn + jax.lax.broadcasted_iota(jnp.int32, (tn, te), 0)
    onehot = (dst_ref[...] == ids).astype(jnp.float32)
    acc_ref[...] += jnp.dot(onehot, m2, preferred_element_type=jnp.float32)
    deg_ref[...] += jnp.sum(onehot, axis=1, keepdims=True)

    @pl.when(k == pl.num_programs(1) - 1)
    def _():
        x2 = acc_ref[...] / jnp.maximum(deg_ref[...], 1.0)
        a_ref[...] = jnp.dot(x2, w1h_ref[...],
                             preferred_element_type=jnp.float32) + bh_ref[...]
        b_out_ref[...] = jnp.dot(x2, w2h_ref[...],
                                 preferred_element_type=jnp.float32)


def _agg2(dst_row, m2, w1h, w2h, bh, *, n_pad, tn, e_pad, te):
    h = w1h.shape[1]
    return pl.pallas_call(
        functools.partial(_agg2_kernel, tn=tn),
        out_shape=(jax.ShapeDtypeStruct((n_pad, h), jnp.float32),
                   jax.ShapeDtypeStruct((n_pad, h), jnp.float32)),
        grid=(n_pad // tn, e_pad // te),
        in_specs=[
            pl.BlockSpec((1, te), lambda i, k: (0, k)),
            pl.BlockSpec((te, h), lambda i, k: (k, 0)),
            pl.BlockSpec((h, h), lambda i, k: (0, 0)),
            pl.BlockSpec((h, h), lambda i, k: (0, 0)),
            pl.BlockSpec((1, h), lambda i, k: (0, 0)),
        ],
        out_specs=(pl.BlockSpec((tn, h), lambda i, k: (i, 0)),
                   pl.BlockSpec((tn, h), lambda i, k: (i, 0))),
        scratch_shapes=[pltpu.VMEM((tn, h), jnp.float32),
                        pltpu.VMEM((tn, 1), jnp.float32)],
        compiler_params=pltpu.CompilerParams(
            dimension_semantics=("parallel", "arbitrary")),
    )(dst_row, m2, w1h, w2h, bh.reshape(1, h))


def _head_kernel(g_ref, s_ref, y_ref, bc_ref, pred_ref, part_ref, *, sw):
    hid = jnp.maximum(g_ref[...], 0.0)
    logits = jnp.dot(hid, s_ref[...], preferred_element_type=jnp.float32)
    logits = logits + bc_ref[...]
    p = jax.nn.sigmoid(logits)
    pred_ref[...] = p

    y = y_ref[...]
    w = jnp.where(p > 0.5, jnp.float32(sw), jnp.float32(1.0))
    log_p = jnp.maximum(jnp.log(p), -100.0)
    log_1mp = jnp.maximum(jnp.log(1.0 - p), -100.0)
    bce = w * -(y * log_p + (1.0 - y) * log_1mp)

    p2 = p * p
    rows = jnp.concatenate([
        jnp.sum(p, axis=0, keepdims=True),
        jnp.sum(p2, axis=0, keepdims=True),
        jnp.sum(p2 * p, axis=0, keepdims=True),
        jnp.sum(p2 * p2, axis=0, keepdims=True),
        jnp.sum(bce, axis=0, keepdims=True),
        jnp.zeros((3, 128), jnp.float32),
    ], axis=0)
    part_ref[...] = rows.reshape(1, 8, 128)


def _head(g2d, s_mat, y2d, bc, *, rows, tl, sw):
    n_tiles = rows // tl
    lanes = g2d.shape[1]
    return pl.pallas_call(
        functools.partial(_head_kernel, sw=sw),
        out_shape=(jax.ShapeDtypeStruct((rows, 128), jnp.float32),
                   jax.ShapeDtypeStruct((n_tiles, 8, 128), jnp.float32)),
        grid=(n_tiles,),
        in_specs=[
            pl.BlockSpec((tl, lanes), lambda i: (i, 0)),
            pl.BlockSpec((lanes, 128), lambda i: (0, 0)),
            pl.BlockSpec((tl, 128), lambda i: (i, 0)),
            pl.BlockSpec((1, 1), lambda i: (0, 0)),
        ],
        out_specs=(pl.BlockSpec((tl, 128), lambda i: (i, 0)),
                   pl.BlockSpec((1, 8, 128), lambda i: (i, 0, 0))),
        compiler_params=pltpu.CompilerParams(dimension_semantics=("parallel",)),
    )(g2d, s_mat, y2d, bc)


def _loss_kernel(part_ref, loss_ref, *, n, reg):
    v = jnp.sum(part_ref[...], axis=0)
    s = jnp.sum(v, axis=1, keepdims=True)
    sp, sp2, sp3, sp4, sbce = (s[0:1, :], s[1:2, :], s[2:3, :], s[3:4, :],
                               s[4:5, :])
    nf = jnp.float32(n)
    mu = sp / nf
    mu2 = mu * mu
    m4 = (sp4 / nf - 4.0 * mu * (sp3 / nf)
          + 6.0 * mu2 * (sp2 / nf) - 3.0 * mu2 * mu2)
    m4 = jnp.maximum(m4, 0.0)
    loss_ref[...] = sbce / nf - jnp.sqrt(jnp.sqrt(m4)) * reg


def _finalize(part, *, n, reg):
    n_tiles = part.shape[0]
    return pl.pallas_call(
        functools.partial(_loss_kernel, n=n, reg=reg),
        out_shape=jax.ShapeDtypeStruct((1, 1), jnp.float32),
        grid=(1,),
        in_specs=[pl.BlockSpec((n_tiles, 8, 128), lambda i: (0, 0, 0))],
        out_specs=pl.BlockSpec((1, 1), lambda i: (0, 0)),
    )(part)


def kernel(x, edge_index, edge_label_index, edge_type,
           convN1_w, convN1_b, convE1_w, convE1_b,
           convN2_w, convN2_b, convE2_w, convE2_b, cls_w, cls_b):
    x = x.astype(jnp.float32)
    n_nodes, f_in = x.shape
    n_edges = edge_index.shape[1]
    n_labels = edge_label_index.shape[1]
    h = _HIDDEN

    src, dst = edge_index[0], edge_index[1]
    lsrc, ldst = edge_label_index[0], edge_label_index[1]

    ea0 = (x[dst] - x[src]).astype(jnp.float32)

    n_pad = _round_up(n_nodes, 8)
    tn = _pick_tile(n_pad, 512)
    e_pad = _round_up(n_edges, 128)
    te = _pick_tile(e_pad, 1024)

    dst_row = jnp.full((1, e_pad), -1, jnp.int32).at[0, :n_edges].set(
        dst.astype(jnp.int32))
    ea0p = _pad_rows(ea0, e_pad)

    we1 = convN1_w[f_in:2 * f_in]
    X1 = _agg1(dst_row, ea0p, we1, convN1_b,
               n_pad=n_pad, tn=tn, e_pad=e_pad, te=te)[:n_nodes]

    xs1 = _pad_rows(X1[src], e_pad)
    xd1 = _pad_rows(X1[dst], e_pad)

    w1, w2, w3 = convE1_w[:h], convE1_w[h:2 * h], convE1_w[2 * h:2 * h + f_in]
    wx2, we2 = convN2_w[:h], convN2_w[h:2 * h]
    w1h, w2h, w3h = convE2_w[:h], convE2_w[h:2 * h], convE2_w[2 * h:3 * h]
    te_b = _pick_tile(e_pad, 2048)
    M2, P = _edge_pass(xs1, xd1, ea0p, w1, w2, w3, convE1_b,
                       wx2, we2, convN2_b, w3h, e_pad=e_pad, te=te_b)

    a_tab, b_tab = _agg2(dst_row, M2, w1h, w2h, convE2_b,
                         n_pad=n_pad, tn=tn, e_pad=e_pad, te=te)
    a_tab, b_tab = a_tab[:n_nodes], b_tab[:n_nodes]

    key = src.astype(jnp.int32) * n_nodes + dst.astype(jnp.int32)
    order = jnp.argsort(key)
    skey = key[order]
    ps = P[:n_edges][order]
    csum = jnp.concatenate(
        [jnp.zeros((1, h), jnp.float32), jnp.cumsum(ps, axis=0)], axis=0)
    is_start = jnp.concatenate(
        [jnp.ones((1,), jnp.bool_), skey[1:] != skey[:-1]])
    eidx = jnp.arange(n_edges, dtype=jnp.int32)
    run_start = jax.lax.cummax(jnp.where(is_start, eidx, 0))
    vrun = jnp.concatenate(
        [jnp.zeros((1, h), jnp.float32), csum[1:] - csum[run_start]], axis=0)

    qkey = lsrc.astype(jnp.int32) * n_nodes + ldst.astype(jnp.int32)
    liota = jnp.arange(n_labels, dtype=jnp.int32)
    sq, sidx = jax.lax.sort((qkey, liota), num_keys=1)
    pl_pos = jnp.searchsorted(sq, skey, side="left")
    pr_pos = jnp.searchsorted(sq, skey, side="right")
    packed = (jnp.zeros((n_labels + 1,), jnp.int32)
              .at[pl_pos].add(65537)
              .at[pr_pos].add(-65536))
    acc = jnp.cumsum(packed)[:n_labels]
    hi_sorted = acc & 0xFFFF
    present = (acc >> 16) > 0
    hi_m_sorted = jnp.where(present, hi_sorted, 0)
    hi_m = jnp.zeros((n_labels,), jnp.int32).at[sidx].set(hi_m_sorted)

    g = a_tab[lsrc] + b_tab[ldst] + vrun[hi_m]
    rows = n_labels // 128
    g2d = g.reshape(rows, 128 * h)
    y2d = edge_type.astype(jnp.float32).reshape(rows, 128)

    wc = cls_w.reshape(h)
    lane_ids = jnp.arange(128 * h, dtype=jnp.int32)
    s_mat = (jnp.tile(wc, 128)[:, None]
             * (lane_ids[:, None] // h
                == jnp.arange(128, dtype=jnp.int32)[None, :]
                ).astype(jnp.float32))

    tl = _pick_tile(rows, 256)
    pred2d, part = _head(g2d, s_mat, y2d, cls_b.reshape(1, 1),
                         rows=rows, tl=tl, sw=float(_SIMILAR_WEIGHT))

    loss = _finalize(part, n=n_labels, reg=float(_REGULARIZER))
    return pred2d.reshape(n_labels), loss[0, 0]


# speedup vs baseline: 23.8301x; 1.3733x over previous
import functools

import jax
import jax.numpy as jnp
from jax.experimental import pallas as pl
from jax.experimental.pallas import tpu as pltpu

_HIDDEN = 32
_SIMILAR_WEIGHT = 1.0
_REGULARIZER = 0.1


def _round_up(n, m):
    return ((n + m - 1) // m) * m


def _pick_tile(padded, want):
    return want if padded % want == 0 else padded


def _pad_rows(a, rows):
    pad = rows - a.shape[0]
    if pad == 0:
        return a
    return jnp.pad(a, ((0, pad), (0, 0)))


def _agg1_kernel(dst_ref, ea_ref, we_ref, b_ref, out_ref, acc_ref, deg_ref,
                 *, tn):
    i = pl.program_id(0)
    k = pl.program_id(1)

    @pl.when(k == 0)
    def _():
        acc_ref[...] = jnp.zeros_like(acc_ref)
        deg_ref[...] = jnp.zeros_like(deg_ref)

    msg = jnp.dot(ea_ref[...], we_ref[...], preferred_element_type=jnp.float32)
    msg = jnp.maximum(msg + b_ref[...], 0.0)
    te = msg.shape[0]
    ids = i * tn + jax.lax.broadcasted_iota(jnp.int32, (tn, te), 0)
    onehot = (dst_ref[...] == ids).astype(jnp.float32)
    acc_ref[...] += jnp.dot(onehot, msg, preferred_element_type=jnp.float32)
    deg_ref[...] += jnp.sum(onehot, axis=1, keepdims=True)

    @pl.when(k == pl.num_programs(1) - 1)
    def _():
        out_ref[...] = acc_ref[...] / jnp.maximum(deg_ref[...], 1.0)


def _agg1(dst_row, ea, we, b, *, n_pad, tn, e_pad, te):
    h = we.shape[1]
    fe = ea.shape[1]
    return pl.pallas_call(
        functools.partial(_agg1_kernel, tn=tn),
        out_shape=jax.ShapeDtypeStruct((n_pad, h), jnp.float32),
        grid=(n_pad // tn, e_pad // te),
        in_specs=[
            pl.BlockSpec((1, te), lambda i, k: (0, k)),
            pl.BlockSpec((te, fe), lambda i, k: (k, 0)),
            pl.BlockSpec((fe, h), lambda i, k: (0, 0)),
            pl.BlockSpec((1, h), lambda i, k: (0, 0)),
        ],
        out_specs=pl.BlockSpec((tn, h), lambda i, k: (i, 0)),
        scratch_shapes=[pltpu.VMEM((tn, h), jnp.float32),
                        pltpu.VMEM((tn, 1), jnp.float32)],
        compiler_params=pltpu.CompilerParams(
            dimension_semantics=("parallel", "arbitrary")),
    )(dst_row, ea, we, b.reshape(1, h))


def _edge_kernel(xs_ref, xd_ref, ea_ref, w1_ref, w2_ref, w3_ref, b_ref,
                 wx2_ref, we2_ref, b2_ref, w3h_ref, m2_ref, p_ref):
    ea1 = jnp.dot(xs_ref[...], w1_ref[...], preferred_element_type=jnp.float32)
    ea1 = ea1 + jnp.dot(xd_ref[...], w2_ref[...],
                        preferred_element_type=jnp.float32)
    ea1 = ea1 + jnp.dot(ea_ref[...], w3_ref[...],
                        preferred_element_type=jnp.float32)
    ea1 = jnp.maximum(ea1 + b_ref[...], 0.0)
    m2 = jnp.dot(xs_ref[...], wx2_ref[...], preferred_element_type=jnp.float32)
    m2 = m2 + jnp.dot(ea1, we2_ref[...], preferred_element_type=jnp.float32)
    m2_ref[...] = jnp.maximum(m2 + b2_ref[...], 0.0)
    p_ref[...] = jnp.dot(ea1, w3h_ref[...], preferred_element_type=jnp.float32)


def _edge_pass(xs, xd, ea, w1, w2, w3, b, wx2, we2, b2, w3h, *, e_pad, te):
    h = w1.shape[1]
    fn = xs.shape[1]
    fe = ea.shape[1]
    return pl.pallas_call(
        _edge_kernel,
        out_shape=(jax.ShapeDtypeStruct((e_pad, h), jnp.float32),
                   jax.ShapeDtypeStruct((e_pad, h), jnp.float32)),
        grid=(e_pad // te,),
        in_specs=[
            pl.BlockSpec((te, fn), lambda i: (i, 0)),
            pl.BlockSpec((te, fn), lambda i: (i, 0)),
            pl.BlockSpec((te, fe), lambda i: (i, 0)),
            pl.BlockSpec((fn, h), lambda i: (0, 0)),
            pl.BlockSpec((fn, h), lambda i: (0, 0)),
            pl.BlockSpec((fe, h), lambda i: (0, 0)),
            pl.BlockSpec((1, h), lambda i: (0, 0)),
            pl.BlockSpec((h, h), lambda i: (0, 0)),
            pl.BlockSpec((h, h), lambda i: (0, 0)),
            pl.BlockSpec((1, h), lambda i: (0, 0)),
            pl.BlockSpec((h, h), lambda i: (0, 0)),
        ],
        out_specs=(pl.BlockSpec((te, h), lambda i: (i, 0)),
                   pl.BlockSpec((te, h), lambda i: (i, 0))),
        compiler_params=pltpu.CompilerParams(dimension_semantics=("parallel",)),
    )(xs, xd, ea, w1, w2, w3, b.reshape(1, h), wx2, we2, b2.reshape(1, h), w3h)


def _agg2_kernel(dst_ref, m2_ref, w1h_ref, w2h_ref, bh_ref,
                 a_ref, b_out_ref, acc_ref, deg_ref, *, tn):
    i = pl.program_id(0)
    k = pl.program_id(1)

    @pl.when(k == 0)
    def _():
        acc_ref[...] = jnp.zeros_like(acc_ref)
        deg_ref[...] = jnp.zeros_like(deg_ref)

    m2 = m2_ref[...]
    te = m2.shape[0]
    ids = i * tn + jax.lax.broadcasted_iota(jnp.int32, (tn, te), 0)
    onehot = (dst_ref[...] == ids).astype(jnp.float32)
    acc_ref[...] += jnp.dot(onehot, m2, preferred_element_type=jnp.float32)
    deg_ref[...] += jnp.sum(onehot, axis=1, keepdims=True)

    @pl.when(k == pl.num_programs(1) - 1)
    def _():
        x2 = acc_ref[...] / jnp.maximum(deg_ref[...], 1.0)
        a_ref[...] = jnp.dot(x2, w1h_ref[...],
                             preferred_element_type=jnp.float32) + bh_ref[...]
        b_out_ref[...] = jnp.dot(x2, w2h_ref[...],
                                 preferred_element_type=jnp.float32)


def _agg2(dst_row, m2, w1h, w2h, bh, *, n_pad, tn, e_pad, te):
    h = w1h.shape[1]
    return pl.pallas_call(
        functools.partial(_agg2_kernel, tn=tn),
        out_shape=(jax.ShapeDtypeStruct((n_pad, h), jnp.float32),
                   jax.ShapeDtypeStruct((n_pad, h), jnp.float32)),
        grid=(n_pad // tn, e_pad // te),
        in_specs=[
            pl.BlockSpec((1, te), lambda i, k: (0, k)),
            pl.BlockSpec((te, h), lambda i, k: (k, 0)),
            pl.BlockSpec((h, h), lambda i, k: (0, 0)),
            pl.BlockSpec((h, h), lambda i, k: (0, 0)),
            pl.BlockSpec((1, h), lambda i, k: (0, 0)),
        ],
        out_specs=(pl.BlockSpec((tn, h), lambda i, k: (i, 0)),
                   pl.BlockSpec((tn, h), lambda i, k: (i, 0))),
        scratch_shapes=[pltpu.VMEM((tn, h), jnp.float32),
                        pltpu.VMEM((tn, 1), jnp.float32)],
        compiler_params=pltpu.CompilerParams(
            dimension_semantics=("parallel", "arbitrary")),
    )(dst_row, m2, w1h, w2h, bh.reshape(1, h))


def _head_kernel(sq_ref, ct_ref, at_ref, bt_ref, wc_ref, bc_ref,
                 pred_ref, part_ref, *, n_nodes, kshift, sw):
    sqv = sq_ref[...]
    lsrc = sqv >> (kshift + 1)
    ldst = (sqv >> 1) & ((1 << kshift) - 1)
    tlq = sqv.shape[1]
    subl = jax.lax.broadcasted_iota(jnp.int32, (n_nodes, tlq), 0)
    oh_a = (subl == lsrc).astype(jnp.bfloat16)
    oh_b = (subl == ldst).astype(jnp.bfloat16)
    gt = jnp.dot(at_ref[...], oh_a, preferred_element_type=jnp.float32)
    gt = gt + jnp.dot(bt_ref[...], oh_b, preferred_element_type=jnp.float32)
    gt = gt + ct_ref[...]
    hh = jnp.maximum(gt, 0.0)
    logits = jnp.sum(hh * wc_ref[...], axis=0, keepdims=True) + bc_ref[...]
    p = jax.nn.sigmoid(logits)
    pred_ref[...] = p

    y = (sqv & 1).astype(jnp.float32)
    w = jnp.where(p > 0.5, jnp.float32(sw), jnp.float32(1.0))
    log_p = jnp.maximum(jnp.log(p), -100.0)
    log_1mp = jnp.maximum(jnp.log(1.0 - p), -100.0)
    bce = w * -(y * log_p + (1.0 - y) * log_1mp)

    p2 = p * p
    sp = jnp.sum(p)
    sp2 = jnp.sum(p2)
    sp3 = jnp.sum(p2 * p)
    sp4 = jnp.sum(p2 * p2)
    sb = jnp.sum(bce)
    lane = jax.lax.broadcasted_iota(jnp.int32, (1, 1, 128), 2)
    row = (jnp.where(lane == 0, sp, 0.0) + jnp.where(lane == 1, sp2, 0.0)
           + jnp.where(lane == 2, sp3, 0.0) + jnp.where(lane == 3, sp4, 0.0)
           + jnp.where(lane == 4, sb, 0.0))
    part_ref[...] = row


def _head(sq_row, ct, a_tabT, b_tabT, wc_col, bc, *, n_labels, tlq,
          n_nodes, kshift, sw):
    n_tiles = n_labels // tlq
    h = ct.shape[0]
    return pl.pallas_call(
        functools.partial(_head_kernel, n_nodes=n_nodes, kshift=kshift,
                          sw=sw),
        out_shape=(jax.ShapeDtypeStruct((1, n_labels), jnp.float32),
                   jax.ShapeDtypeStruct((n_tiles, 1, 128), jnp.float32)),
        grid=(n_tiles,),
        in_specs=[
            pl.BlockSpec((1, tlq), lambda i: (0, i)),
            pl.BlockSpec((h, tlq), lambda i: (0, i)),
            pl.BlockSpec((h, n_nodes), lambda i: (0, 0)),
            pl.BlockSpec((h, n_nodes), lambda i: (0, 0)),
            pl.BlockSpec((h, 1), lambda i: (0, 0)),
            pl.BlockSpec((1, 1), lambda i: (0, 0)),
        ],
        out_specs=(pl.BlockSpec((1, tlq), lambda i: (0, i)),
                   pl.BlockSpec((1, 1, 128), lambda i: (i, 0, 0))),
        compiler_params=pltpu.CompilerParams(dimension_semantics=("parallel",)),
    )(sq_row, ct, a_tabT, b_tabT, wc_col, bc)


def _loss_kernel(part_ref, loss_ref, *, n, reg):
    v = jnp.sum(part_ref[...], axis=0)
    sp, sp2, sp3, sp4, sbce = (v[:, 0:1], v[:, 1:2], v[:, 2:3], v[:, 3:4],
                               v[:, 4:5])
    nf = jnp.float32(n)
    mu = sp / nf
    mu2 = mu * mu
    m4 = (sp4 / nf - 4.0 * mu * (sp3 / nf)
          + 6.0 * mu2 * (sp2 / nf) - 3.0 * mu2 * mu2)
    m4 = jnp.maximum(m4, 0.0)
    loss_ref[...] = sbce / nf - jnp.sqrt(jnp.sqrt(m4)) * reg


def _finalize(part, *, n, reg):
    n_tiles = part.shape[0]
    return pl.pallas_call(
        functools.partial(_loss_kernel, n=n, reg=reg),
        out_shape=jax.ShapeDtypeStruct((1, 1), jnp.float32),
        grid=(1,),
        in_specs=[pl.BlockSpec((n_tiles, 1, 128), lambda i: (0, 0, 0))],
        out_specs=pl.BlockSpec((1, 1), lambda i: (0, 0)),
    )(part)


def kernel(x, edge_index, edge_label_index, edge_type,
           convN1_w, convN1_b, convE1_w, convE1_b,
           convN2_w, convN2_b, convE2_w, convE2_b, cls_w, cls_b):
    x = x.astype(jnp.float32)
    n_nodes, f_in = x.shape
    n_edges = edge_index.shape[1]
    n_labels = edge_label_index.shape[1]
    h = _HIDDEN

    src, dst = edge_index[0], edge_index[1]
    lsrc, ldst = edge_label_index[0], edge_label_index[1]

    ea0 = (x[dst] - x[src]).astype(jnp.float32)

    n_pad = _round_up(n_nodes, 8)
    tn = _pick_tile(n_pad, 512)
    e_pad = _round_up(n_edges, 128)
    te = _pick_tile(e_pad, 1024)

    dst_row = jnp.full((1, e_pad), -1, jnp.int32).at[0, :n_edges].set(
        dst.astype(jnp.int32))
    ea0p = _pad_rows(ea0, e_pad)

    we1 = convN1_w[f_in:2 * f_in]
    X1 = _agg1(dst_row, ea0p, we1, convN1_b,
               n_pad=n_pad, tn=tn, e_pad=e_pad, te=te)[:n_nodes]

    xs1 = _pad_rows(X1[src], e_pad)
    xd1 = _pad_rows(X1[dst], e_pad)

    w1, w2, w3 = convE1_w[:h], convE1_w[h:2 * h], convE1_w[2 * h:2 * h + f_in]
    wx2, we2 = convN2_w[:h], convN2_w[h:2 * h]
    w1h, w2h, w3h = convE2_w[:h], convE2_w[h:2 * h], convE2_w[2 * h:3 * h]
    te_b = _pick_tile(e_pad, 2048)
    M2, P = _edge_pass(xs1, xd1, ea0p, w1, w2, w3, convE1_b,
                       wx2, we2, convN2_b, w3h, e_pad=e_pad, te=te_b)

    a_tab, b_tab = _agg2(dst_row, M2, w1h, w2h, convE2_b,
                         n_pad=n_pad, tn=tn, e_pad=e_pad, te=te)
    a_tab, b_tab = a_tab[:n_nodes], b_tab[:n_nodes]

    kshift = max(int(n_nodes - 1).bit_length(), 1)
    kmul = 1 << kshift
    key = src.astype(jnp.int32) * kmul + dst.astype(jnp.int32)
    order = jnp.argsort(key)
    skey = key[order]
    ps = P[:n_edges][order]
    csum = jnp.concatenate(
        [jnp.zeros((1, h), jnp.float32), jnp.cumsum(ps, axis=0)], axis=0)
    is_start = jnp.concatenate(
        [jnp.ones((1,), jnp.bool_), skey[1:] != skey[:-1]])
    eidx = jnp.arange(n_edges, dtype=jnp.int32)
    run_start = jax.lax.cummax(jnp.where(is_start, eidx, 0))
    vrun = jnp.concatenate(
        [jnp.zeros((1, h), jnp.float32), csum[1:] - csum[run_start]], axis=0)

    qkey = lsrc.astype(jnp.int32) * kmul + ldst.astype(jnp.int32)
    key2 = (qkey << 1) | edge_type.astype(jnp.int32)
    liota = jnp.arange(n_labels, dtype=jnp.int32)
    sq, sidx = jax.lax.sort((key2, liota), num_keys=1)
    sqk = sq >> 1
    pl_pos = jnp.searchsorted(sqk, skey, side="left")
    pr_pos = jnp.searchsorted(sqk, skey, side="right")
    packed = (jnp.zeros((n_labels + 1,), jnp.int32)
              .at[pl_pos].add(65537)
              .at[pr_pos].add(-65536))
    acc = jnp.cumsum(packed)[:n_labels]
    hi_sorted = acc & 0xFFFF
    present = (acc >> 16) > 0
    hi_m_sorted = jnp.where(present, hi_sorted, 0)

    ct = vrun.T[:, hi_m_sorted]

    a_tabT = _pad_rows(a_tab, n_pad).T.astype(jnp.bfloat16)
    b_tabT = _pad_rows(b_tab, n_pad).T.astype(jnp.bfloat16)
    tlq = _pick_tile(n_labels, 512)
    pred_row, part = _head(
        sq.reshape(1, n_labels), ct, a_tabT, b_tabT,
        cls_w.reshape(h, 1), cls_b.reshape(1, 1),
        n_labels=n_labels, tlq=tlq, n_nodes=n_pad, kshift=kshift,
        sw=float(_SIMILAR_WEIGHT))

    loss = _finalize(part, n=n_labels, reg=float(_REGULARIZER))
    edge_pred = jnp.zeros((n_labels,), jnp.float32).at[sidx].set(pred_row[0])
    return edge_pred, loss[0, 0]


# speedup vs baseline: 33.0303x; 1.3861x over previous
import functools

import jax
import jax.numpy as jnp
from jax.experimental import pallas as pl
from jax.experimental.pallas import tpu as pltpu

_HIDDEN = 32
_SIMILAR_WEIGHT = 1.0
_REGULARIZER = 0.1


def _round_up(n, m):
    return ((n + m - 1) // m) * m


def _pick_tile(padded, want):
    return want if padded % want == 0 else padded


def _pad_rows(a, rows):
    pad = rows - a.shape[0]
    if pad == 0:
        return a
    return jnp.pad(a, ((0, pad), (0, 0)))


def _agg1_kernel(dst_ref, ea_ref, we_ref, b_ref, out_ref, acc_ref, deg_ref,
                 *, tn):
    i = pl.program_id(0)
    k = pl.program_id(1)

    @pl.when(k == 0)
    def _():
        acc_ref[...] = jnp.zeros_like(acc_ref)
        deg_ref[...] = jnp.zeros_like(deg_ref)

    msg = jnp.dot(ea_ref[...], we_ref[...], preferred_element_type=jnp.float32)
    msg = jnp.maximum(msg + b_ref[...], 0.0)
    te = msg.shape[0]
    ids = i * tn + jax.lax.broadcasted_iota(jnp.int32, (tn, te), 0)
    onehot = (dst_ref[...] == ids).astype(jnp.float32)
    acc_ref[...] += jnp.dot(onehot, msg, preferred_element_type=jnp.float32)
    deg_ref[...] += jnp.sum(onehot, axis=1, keepdims=True)

    @pl.when(k == pl.num_programs(1) - 1)
    def _():
        out_ref[...] = acc_ref[...] / jnp.maximum(deg_ref[...], 1.0)


def _agg1(dst_row, ea, we, b, *, n_pad, tn, e_pad, te):
    h = we.shape[1]
    fe = ea.shape[1]
    return pl.pallas_call(
        functools.partial(_agg1_kernel, tn=tn),
        out_shape=jax.ShapeDtypeStruct((n_pad, h), jnp.float32),
        grid=(n_pad // tn, e_pad // te),
        in_specs=[
            pl.BlockSpec((1, te), lambda i, k: (0, k)),
            pl.BlockSpec((te, fe), lambda i, k: (k, 0)),
            pl.BlockSpec((fe, h), lambda i, k: (0, 0)),
            pl.BlockSpec((1, h), lambda i, k: (0, 0)),
        ],
        out_specs=pl.BlockSpec((tn, h), lambda i, k: (i, 0)),
        scratch_shapes=[pltpu.VMEM((tn, h), jnp.float32),
                        pltpu.VMEM((tn, 1), jnp.float32)],
        compiler_params=pltpu.CompilerParams(
            dimension_semantics=("parallel", "arbitrary")),
    )(dst_row, ea, we, b.reshape(1, h))


def _edge_kernel(xs_ref, xd_ref, ea_ref, w1_ref, w2_ref, w3_ref, b_ref,
                 wx2_ref, we2_ref, b2_ref, w3h_ref, m2_ref, p_ref):
    ea1 = jnp.dot(xs_ref[...], w1_ref[...], preferred_element_type=jnp.float32)
    ea1 = ea1 + jnp.dot(xd_ref[...], w2_ref[...],
                        preferred_element_type=jnp.float32)
    ea1 = ea1 + jnp.dot(ea_ref[...], w3_ref[...],
                        preferred_element_type=jnp.float32)
    ea1 = jnp.maximum(ea1 + b_ref[...], 0.0)
    m2 = jnp.dot(xs_ref[...], wx2_ref[...], preferred_element_type=jnp.float32)
    m2 = m2 + jnp.dot(ea1, we2_ref[...], preferred_element_type=jnp.float32)
    m2_ref[...] = jnp.maximum(m2 + b2_ref[...], 0.0)
    p_ref[...] = jnp.dot(ea1, w3h_ref[...], preferred_element_type=jnp.float32)


def _edge_pass(xs, xd, ea, w1, w2, w3, b, wx2, we2, b2, w3h, *, e_pad, te):
    h = w1.shape[1]
    fn = xs.shape[1]
    fe = ea.shape[1]
    return pl.pallas_call(
        _edge_kernel,
        out_shape=(jax.ShapeDtypeStruct((e_pad, h), jnp.float32),
                   jax.ShapeDtypeStruct((e_pad, h), jnp.float32)),
        grid=(e_pad // te,),
        in_specs=[
            pl.BlockSpec((te, fn), lambda i: (i, 0)),
            pl.BlockSpec((te, fn), lambda i: (i, 0)),
            pl.BlockSpec((te, fe), lambda i: (i, 0)),
            pl.BlockSpec((fn, h), lambda i: (0, 0)),
            pl.BlockSpec((fn, h), lambda i: (0, 0)),
            pl.BlockSpec((fe, h), lambda i: (0, 0)),
            pl.BlockSpec((1, h), lambda i: (0, 0)),
            pl.BlockSpec((h, h), lambda i: (0, 0)),
            pl.BlockSpec((h, h), lambda i: (0, 0)),
            pl.BlockSpec((1, h), lambda i: (0, 0)),
            pl.BlockSpec((h, h), lambda i: (0, 0)),
        ],
        out_specs=(pl.BlockSpec((te, h), lambda i: (i, 0)),
                   pl.BlockSpec((te, h), lambda i: (i, 0))),
        compiler_params=pltpu.CompilerParams(dimension_semantics=("parallel",)),
    )(xs, xd, ea, w1, w2, w3, b.reshape(1, h), wx2, we2, b2.reshape(1, h), w3h)


def _agg2_kernel(dst_ref, m2_ref, w1h_ref, w2h_ref, bh_ref,
                 a_ref, b_out_ref, acc_ref, deg_ref, *, tn):
    i = pl.program_id(0)
    k = pl.program_id(1)

    @pl.when(k == 0)
    def _():
        acc_ref[...] = jnp.zeros_like(acc_ref)
        deg_ref[...] = jnp.zeros_like(deg_ref)

    m2 = m2_ref[...]
    te = m2.shape[0]
    ids = i * tn + jax.lax.broadcasted_iota(jnp.int32, (tn, te), 0)
    onehot = (dst_ref[...] == ids).astype(jnp.float32)
    acc_ref[...] += jnp.dot(onehot, m2, preferred_element_type=jnp.float32)
    deg_ref[...] += jnp.sum(onehot, axis=1, keepdims=True)

    @pl.when(k == pl.num_programs(1) - 1)
    def _():
        x2 = acc_ref[...] / jnp.maximum(deg_ref[...], 1.0)
        a_ref[...] = jnp.dot(x2, w1h_ref[...],
                             preferred_element_type=jnp.float32) + bh_ref[...]
        b_out_ref[...] = jnp.dot(x2, w2h_ref[...],
                                 preferred_element_type=jnp.float32)


def _agg2(dst_row, m2, w1h, w2h, bh, *, n_pad, tn, e_pad, te):
    h = w1h.shape[1]
    return pl.pallas_call(
        functools.partial(_agg2_kernel, tn=tn),
        out_shape=(jax.ShapeDtypeStruct((n_pad, h), jnp.float32),
                   jax.ShapeDtypeStruct((n_pad, h), jnp.float32)),
        grid=(n_pad // tn, e_pad // te),
        in_specs=[
            pl.BlockSpec((1, te), lambda i, k: (0, k)),
            pl.BlockSpec((te, h), lambda i, k: (k, 0)),
            pl.BlockSpec((h, h), lambda i, k: (0, 0)),
            pl.BlockSpec((h, h), lambda i, k: (0, 0)),
            pl.BlockSpec((1, h), lambda i, k: (0, 0)),
        ],
        out_specs=(pl.BlockSpec((tn, h), lambda i, k: (i, 0)),
                   pl.BlockSpec((tn, h), lambda i, k: (i, 0))),
        scratch_shapes=[pltpu.VMEM((tn, h), jnp.float32),
                        pltpu.VMEM((tn, 1), jnp.float32)],
        compiler_params=pltpu.CompilerParams(
            dimension_semantics=("parallel", "arbitrary")),
    )(dst_row, m2, w1h, w2h, bh.reshape(1, h))


def _head_kernel(sq_ref, acc_ref, at_ref, bt_ref, vt_ref, wc_ref, bc_ref,
                 pred_ref, part_ref, *, n_nodes, kshift, sw):
    sqv = sq_ref[...]
    lsrc = sqv >> (kshift + 1)
    ldst = (sqv >> 1) & ((1 << kshift) - 1)
    tlq = sqv.shape[1]
    subl = jax.lax.broadcasted_iota(jnp.int32, (n_nodes, tlq), 0)
    oh_a = (subl == lsrc).astype(jnp.bfloat16)
    oh_b = (subl == ldst).astype(jnp.bfloat16)
    gt = jnp.dot(at_ref[...], oh_a, preferred_element_type=jnp.float32)
    gt = gt + jnp.dot(bt_ref[...], oh_b, preferred_element_type=jnp.float32)

    accv = acc_ref[...]
    hiv = accv & 0xFFFF
    pres = (accv >> 16) > 0
    h = gt.shape[0]
    lo_s = jnp.min(jnp.where(pres, hiv, jnp.int32(1 << 30)))
    hi_s = jnp.max(jnp.where(pres, hiv, jnp.int32(-1)))
    lane128 = jax.lax.broadcasted_iota(jnp.int32, (1, 128), 1)

    def body(a, ct):
        cbase = pl.multiple_of((a >> 7) * 128, 128)
        chunk = vt_ref[:, pl.ds(cbase, 128)]
        lmask = (lane128 == (a & 127)).astype(jnp.float32)
        col = jnp.sum(chunk * lmask, axis=1, keepdims=True)
        qmask = (hiv == a).astype(jnp.float32)
        return ct + col * qmask

    ct = jax.lax.fori_loop(lo_s, hi_s + 1, body,
                           jnp.zeros((h, tlq), jnp.float32))
    gt = gt + ct * pres.astype(jnp.float32)
    hh = jnp.maximum(gt, 0.0)
    logits = jnp.sum(hh * wc_ref[...], axis=0, keepdims=True) + bc_ref[...]
    p = jax.nn.sigmoid(logits)
    pred_ref[...] = p

    y = (sqv & 1).astype(jnp.float32)
    w = jnp.where(p > 0.5, jnp.float32(sw), jnp.float32(1.0))
    log_p = jnp.maximum(jnp.log(p), -100.0)
    log_1mp = jnp.maximum(jnp.log(1.0 - p), -100.0)
    bce = w * -(y * log_p + (1.0 - y) * log_1mp)

    p2 = p * p
    sp = jnp.sum(p)
    sp2 = jnp.sum(p2)
    sp3 = jnp.sum(p2 * p)
    sp4 = jnp.sum(p2 * p2)
    sb = jnp.sum(bce)
    lane = jax.lax.broadcasted_iota(jnp.int32, (1, 1, 128), 2)
    row = (jnp.where(lane == 0, sp, 0.0) + jnp.where(lane == 1, sp2, 0.0)
           + jnp.where(lane == 2, sp3, 0.0) + jnp.where(lane == 3, sp4, 0.0)
           + jnp.where(lane == 4, sb, 0.0))
    part_ref[...] = row


def _head(sq_row, acc_row, a_tabT, b_tabT, vrunT, wc_col, bc, *, n_labels,
          tlq, n_nodes, kshift, sw):
    n_tiles = n_labels // tlq
    h = a_tabT.shape[0]
    ve = vrunT.shape[1]
    return pl.pallas_call(
        functools.partial(_head_kernel, n_nodes=n_nodes, kshift=kshift,
                          sw=sw),
        out_shape=(jax.ShapeDtypeStruct((1, n_labels), jnp.float32),
                   jax.ShapeDtypeStruct((n_tiles, 1, 128), jnp.float32)),
        grid=(n_tiles,),
        in_specs=[
            pl.BlockSpec((1, tlq), lambda i: (0, i)),
            pl.BlockSpec((1, tlq), lambda i: (0, i)),
            pl.BlockSpec((h, n_nodes), lambda i: (0, 0)),
            pl.BlockSpec((h, n_nodes), lambda i: (0, 0)),
            pl.BlockSpec((h, ve), lambda i: (0, 0)),
            pl.BlockSpec((h, 1), lambda i: (0, 0)),
            pl.BlockSpec((1, 1), lambda i: (0, 0)),
        ],
        out_specs=(pl.BlockSpec((1, tlq), lambda i: (0, i)),
                   pl.BlockSpec((1, 1, 128), lambda i: (i, 0, 0))),
        compiler_params=pltpu.CompilerParams(dimension_semantics=("parallel",)),
    )(sq_row, acc_row, a_tabT, b_tabT, vrunT, wc_col, bc)


def _loss_kernel(part_ref, loss_ref, *, n, reg):
    v = jnp.sum(part_ref[...], axis=0)
    sp, sp2, sp3, sp4, sbce = (v[:, 0:1], v[:, 1:2], v[:, 2:3], v[:, 3:4],
                               v[:, 4:5])
    nf = jnp.float32(n)
    mu = sp / nf
    mu2 = mu * mu
    m4 = (sp4 / nf - 4.0 * mu * (sp3 / nf)
          + 6.0 * mu2 * (sp2 / nf) - 3.0 * mu2 * mu2)
    m4 = jnp.maximum(m4, 0.0)
    loss_ref[...] = sbce / nf - jnp.sqrt(jnp.sqrt(m4)) * reg


def _finalize(part, *, n, reg):
    n_tiles = part.shape[0]
    return pl.pallas_call(
        functools.partial(_loss_kernel, n=n, reg=reg),
        out_shape=jax.ShapeDtypeStruct((1, 1), jnp.float32),
        grid=(1,),
        in_specs=[pl.BlockSpec((n_tiles, 1, 128), lambda i: (0, 0, 0))],
        out_specs=pl.BlockSpec((1, 1), lambda i: (0, 0)),
    )(part)


def kernel(x, edge_index, edge_label_index, edge_type,
           convN1_w, convN1_b, convE1_w, convE1_b,
           convN2_w, convN2_b, convE2_w, convE2_b, cls_w, cls_b):
    x = x.astype(jnp.float32)
    n_nodes, f_in = x.shape
    n_edges = edge_index.shape[1]
    n_labels = edge_label_index.shape[1]
    h = _HIDDEN

    src, dst = edge_index[0], edge_index[1]
    lsrc, ldst = edge_label_index[0], edge_label_index[1]

    ea0 = (x[dst] - x[src]).astype(jnp.float32)

    n_pad = _round_up(n_nodes, 8)
    tn = _pick_tile(n_pad, 512)
    e_pad = _round_up(n_edges, 128)
    te = _pick_tile(e_pad, 1024)

    dst_row = jnp.full((1, e_pad), -1, jnp.int32).at[0, :n_edges].set(
        dst.astype(jnp.int32))
    ea0p = _pad_rows(ea0, e_pad)

    we1 = convN1_w[f_in:2 * f_in]
    X1 = _agg1(dst_row, ea0p, we1, convN1_b,
               n_pad=n_pad, tn=tn, e_pad=e_pad, te=te)[:n_nodes]

    xs1 = _pad_rows(X1[src], e_pad)
    xd1 = _pad_rows(X1[dst], e_pad)

    w1, w2, w3 = convE1_w[:h], convE1_w[h:2 * h], convE1_w[2 * h:2 * h + f_in]
    wx2, we2 = convN2_w[:h], convN2_w[h:2 * h]
    w1h, w2h, w3h = convE2_w[:h], convE2_w[h:2 * h], convE2_w[2 * h:3 * h]
    te_b = _pick_tile(e_pad, 2048)
    M2, P = _edge_pass(xs1, xd1, ea0p, w1, w2, w3, convE1_b,
                       wx2, we2, convN2_b, w3h, e_pad=e_pad, te=te_b)

    a_tab, b_tab = _agg2(dst_row, M2, w1h, w2h, convE2_b,
                         n_pad=n_pad, tn=tn, e_pad=e_pad, te=te)
    a_tab, b_tab = a_tab[:n_nodes], b_tab[:n_nodes]

    kshift = max(int(n_nodes - 1).bit_length(), 1)
    kmul = 1 << kshift
    key = src.astype(jnp.int32) * kmul + dst.astype(jnp.int32)
    order = jnp.argsort(key)
    skey = key[order]
    ps = P[:n_edges][order]
    csum = jnp.concatenate(
        [jnp.zeros((1, h), jnp.float32), jnp.cumsum(ps, axis=0)], axis=0)
    is_start = jnp.concatenate(
        [jnp.ones((1,), jnp.bool_), skey[1:] != skey[:-1]])
    eidx = jnp.arange(n_edges, dtype=jnp.int32)
    run_start = jax.lax.cummax(jnp.where(is_start, eidx, 0))
    vrun = jnp.concatenate(
        [jnp.zeros((1, h), jnp.float32), csum[1:] - csum[run_start]], axis=0)

    qkey = lsrc.astype(jnp.int32) * kmul + ldst.astype(jnp.int32)
    key2 = (qkey << 1) | edge_type.astype(jnp.int32)
    liota = jnp.arange(n_labels, dtype=jnp.int32)
    sq, sidx = jax.lax.sort((key2, liota), num_keys=1)
    sqk = sq >> 1
    pl_pos = jnp.searchsorted(sqk, skey, side="left")
    pr_pos = jnp.searchsorted(sqk, skey, side="right")
    packed = (jnp.zeros((n_labels + 1,), jnp.int32)
              .at[pl_pos].add(65537)
              .at[pr_pos].add(-65536))
    acc = jnp.cumsum(packed)[:n_labels]

    a_tabT = _pad_rows(a_tab, n_pad).T.astype(jnp.bfloat16)
    b_tabT = _pad_rows(b_tab, n_pad).T.astype(jnp.bfloat16)
    ve = _round_up(n_edges + 1, 128)
    vrunT = jnp.zeros((h, ve), jnp.float32).at[:, :n_edges + 1].set(vrun.T)
    tlq = _pick_tile(n_labels, 512)
    pred_row, part = _head(
        sq.reshape(1, n_labels), acc.reshape(1, n_labels), a_tabT, b_tabT,
        vrunT, cls_w.reshape(h, 1), cls_b.reshape(1, 1),
        n_labels=n_labels, tlq=tlq, n_nodes=n_pad, kshift=kshift,
        sw=float(_SIMILAR_WEIGHT))

    loss = _finalize(part, n=n_labels, reg=float(_REGULARIZER))
    edge_pred = jnp.zeros((n_labels,), jnp.float32).at[sidx].set(pred_row[0])
    return edge_pred, loss[0, 0]


# speedup vs baseline: 37.5222x; 1.1360x over previous
import functools

import jax
import jax.numpy as jnp
from jax.experimental import pallas as pl
from jax.experimental.pallas import tpu as pltpu

_HIDDEN = 32
_SIMILAR_WEIGHT = 1.0
_REGULARIZER = 0.1


def _round_up(n, m):
    return ((n + m - 1) // m) * m


def _pick_tile(padded, want):
    return want if padded % want == 0 else padded


def _pad_rows(a, rows):
    pad = rows - a.shape[0]
    if pad == 0:
        return a
    return jnp.pad(a, ((0, pad), (0, 0)))


def _agg1_kernel(dst_ref, ea_ref, we_ref, b_ref, out_ref, acc_ref, deg_ref,
                 *, tn):
    i = pl.program_id(0)
    k = pl.program_id(1)

    @pl.when(k == 0)
    def _():
        acc_ref[...] = jnp.zeros_like(acc_ref)
        deg_ref[...] = jnp.zeros_like(deg_ref)

    msg = jnp.dot(ea_ref[...], we_ref[...], preferred_element_type=jnp.float32)
    msg = jnp.maximum(msg + b_ref[...], 0.0)
    te = msg.shape[0]
    ids = i * tn + jax.lax.broadcasted_iota(jnp.int32, (tn, te), 0)
    onehot = (dst_ref[...] == ids).astype(jnp.float32)
    acc_ref[...] += jnp.dot(onehot, msg, preferred_element_type=jnp.float32)
    deg_ref[...] += jnp.sum(onehot, axis=1, keepdims=True)

    @pl.when(k == pl.num_programs(1) - 1)
    def _():
        out_ref[...] = acc_ref[...] / jnp.maximum(deg_ref[...], 1.0)


def _agg1(dst_row, ea, we, b, *, n_pad, tn, e_pad, te):
    h = we.shape[1]
    fe = ea.shape[1]
    return pl.pallas_call(
        functools.partial(_agg1_kernel, tn=tn),
        out_shape=jax.ShapeDtypeStruct((n_pad, h), jnp.float32),
        grid=(n_pad // tn, e_pad // te),
        in_specs=[
            pl.BlockSpec((1, te), lambda i, k: (0, k)),
            pl.BlockSpec((te, fe), lambda i, k: (k, 0)),
            pl.BlockSpec((fe, h), lambda i, k: (0, 0)),
            pl.BlockSpec((1, h), lambda i, k: (0, 0)),
        ],
        out_specs=pl.BlockSpec((tn, h), lambda i, k: (i, 0)),
        scratch_shapes=[pltpu.VMEM((tn, h), jnp.float32),
                        pltpu.VMEM((tn, 1), jnp.float32)],
        compiler_params=pltpu.CompilerParams(
            dimension_semantics=("parallel", "arbitrary")),
    )(dst_row, ea, we, b.reshape(1, h))


def _edge_kernel(xs_ref, xd_ref, ea_ref, w1_ref, w2_ref, w3_ref, b_ref,
                 wx2_ref, we2_ref, b2_ref, w3h_ref, m2_ref, p_ref):
    ea1 = jnp.dot(xs_ref[...], w1_ref[...], preferred_element_type=jnp.float32)
    ea1 = ea1 + jnp.dot(xd_ref[...], w2_ref[...],
                        preferred_element_type=jnp.float32)
    ea1 = ea1 + jnp.dot(ea_ref[...], w3_ref[...],
                        preferred_element_type=jnp.float32)
    ea1 = jnp.maximum(ea1 + b_ref[...], 0.0)
    m2 = jnp.dot(xs_ref[...], wx2_ref[...], preferred_element_type=jnp.float32)
    m2 = m2 + jnp.dot(ea1, we2_ref[...], preferred_element_type=jnp.float32)
    m2_ref[...] = jnp.maximum(m2 + b2_ref[...], 0.0)
    p_ref[...] = jnp.dot(ea1, w3h_ref[...], preferred_element_type=jnp.float32)


def _edge_pass(xs, xd, ea, w1, w2, w3, b, wx2, we2, b2, w3h, *, e_pad, te):
    h = w1.shape[1]
    fn = xs.shape[1]
    fe = ea.shape[1]
    return pl.pallas_call(
        _edge_kernel,
        out_shape=(jax.ShapeDtypeStruct((e_pad, h), jnp.float32),
                   jax.ShapeDtypeStruct((e_pad, h), jnp.float32)),
        grid=(e_pad // te,),
        in_specs=[
            pl.BlockSpec((te, fn), lambda i: (i, 0)),
            pl.BlockSpec((te, fn), lambda i: (i, 0)),
            pl.BlockSpec((te, fe), lambda i: (i, 0)),
            pl.BlockSpec((fn, h), lambda i: (0, 0)),
            pl.BlockSpec((fn, h), lambda i: (0, 0)),
            pl.BlockSpec((fe, h), lambda i: (0, 0)),
            pl.BlockSpec((1, h), lambda i: (0, 0)),
            pl.BlockSpec((h, h), lambda i: (0, 0)),
            pl.BlockSpec((h, h), lambda i: (0, 0)),
            pl.BlockSpec((1, h), lambda i: (0, 0)),
            pl.BlockSpec((h, h), lambda i: (0, 0)),
        ],
        out_specs=(pl.BlockSpec((te, h), lambda i: (i, 0)),
                   pl.BlockSpec((te, h), lambda i: (i, 0))),
        compiler_params=pltpu.CompilerParams(dimension_semantics=("parallel",)),
    )(xs, xd, ea, w1, w2, w3, b.reshape(1, h), wx2, we2, b2.reshape(1, h), w3h)


def _agg2_kernel(dst_ref, m2_ref, w1h_ref, w2h_ref, bh_ref,
                 a_ref, b_out_ref, acc_ref, deg_ref, *, tn):
    i = pl.program_id(0)
    k = pl.program_id(1)

    @pl.when(k == 0)
    def _():
        acc_ref[...] = jnp.zeros_like(acc_ref)
        deg_ref[...] = jnp.zeros_like(deg_ref)

    m2 = m2_ref[...]
    te = m2.shape[0]
    ids = i * tn + jax.lax.broadcasted_iota(jnp.int32, (tn, te), 0)
    onehot = (dst_ref[...] == ids).astype(jnp.float32)
    acc_ref[...] += jnp.dot(onehot, m2, preferred_element_type=jnp.float32)
    deg_ref[...] += jnp.sum(onehot, axis=1, keepdims=True)

    @pl.when(k == pl.num_programs(1) - 1)
    def _():
        x2 = acc_ref[...] / jnp.maximum(deg_ref[...], 1.0)
        a_ref[...] = jnp.dot(x2, w1h_ref[...],
                             preferred_element_type=jnp.float32) + bh_ref[...]
        b_out_ref[...] = jnp.dot(x2, w2h_ref[...],
                                 preferred_element_type=jnp.float32)


def _agg2(dst_row, m2, w1h, w2h, bh, *, n_pad, tn, e_pad, te):
    h = w1h.shape[1]
    return pl.pallas_call(
        functools.partial(_agg2_kernel, tn=tn),
        out_shape=(jax.ShapeDtypeStruct((n_pad, h), jnp.float32),
                   jax.ShapeDtypeStruct((n_pad, h), jnp.float32)),
        grid=(n_pad // tn, e_pad // te),
        in_specs=[
            pl.BlockSpec((1, te), lambda i, k: (0, k)),
            pl.BlockSpec((te, h), lambda i, k: (k, 0)),
            pl.BlockSpec((h, h), lambda i, k: (0, 0)),
            pl.BlockSpec((h, h), lambda i, k: (0, 0)),
            pl.BlockSpec((1, h), lambda i, k: (0, 0)),
        ],
        out_specs=(pl.BlockSpec((tn, h), lambda i, k: (i, 0)),
                   pl.BlockSpec((tn, h), lambda i, k: (i, 0))),
        scratch_shapes=[pltpu.VMEM((tn, h), jnp.float32),
                        pltpu.VMEM((tn, 1), jnp.float32)],
        compiler_params=pltpu.CompilerParams(
            dimension_semantics=("parallel", "arbitrary")),
    )(dst_row, m2, w1h, w2h, bh.reshape(1, h))


def _head_kernel(sq_ref, acc_ref, at_ref, bt_ref, vt_ref, wc_ref, bc_ref,
                 pred_ref, part_ref, *, n_nodes, kshift, sw):
    sqv = sq_ref[...]
    lsrc = sqv >> (kshift + 1)
    ldst = (sqv >> 1) & ((1 << kshift) - 1)
    tlq = sqv.shape[1]
    subl = jax.lax.broadcasted_iota(jnp.int32, (n_nodes, tlq), 0)
    oh_b = (subl == ldst).astype(jnp.bfloat16)
    gt = jnp.dot(bt_ref[...], oh_b, preferred_element_type=jnp.float32)
    h = gt.shape[0]
    lane128 = jax.lax.broadcasted_iota(jnp.int32, (1, 128), 1)

    a_lo = jnp.min(lsrc)
    a_hi = jnp.max(lsrc)

    def body_a(a, ga):
        cbase = pl.multiple_of((a >> 7) * 128, 128)
        chunk = at_ref[:, pl.ds(cbase, 128)]
        lmask = (lane128 == (a & 127)).astype(jnp.float32)
        col = jnp.sum(chunk * lmask, axis=1, keepdims=True)
        return ga + col * (lsrc == a).astype(jnp.float32)

    gt = gt + jax.lax.fori_loop(a_lo, a_hi + 1, body_a,
                                jnp.zeros((h, tlq), jnp.float32))

    accv = acc_ref[...]
    hiv = accv & 0xFFFF
    pres = (accv >> 16) > 0
    lo_s = jnp.min(jnp.where(pres, hiv, jnp.int32(1 << 30)))
    hi_s = jnp.max(jnp.where(pres, hiv, jnp.int32(-1)))

    def body(a, ct):
        cbase = pl.multiple_of((a >> 7) * 128, 128)
        chunk = vt_ref[:, pl.ds(cbase, 128)]
        lmask = (lane128 == (a & 127)).astype(jnp.float32)
        col = jnp.sum(chunk * lmask, axis=1, keepdims=True)
        qmask = (hiv == a).astype(jnp.float32)
        return ct + col * qmask

    ct = jax.lax.fori_loop(lo_s, hi_s + 1, body,
                           jnp.zeros((h, tlq), jnp.float32))
    gt = gt + ct * pres.astype(jnp.float32)
    hh = jnp.maximum(gt, 0.0)
    logits = jnp.sum(hh * wc_ref[...], axis=0, keepdims=True) + bc_ref[...]
    p = jax.nn.sigmoid(logits)
    pred_ref[...] = p

    y = (sqv & 1).astype(jnp.float32)
    w = jnp.where(p > 0.5, jnp.float32(sw), jnp.float32(1.0))
    log_p = jnp.maximum(jnp.log(p), -100.0)
    log_1mp = jnp.maximum(jnp.log(1.0 - p), -100.0)
    bce = w * -(y * log_p + (1.0 - y) * log_1mp)

    p2 = p * p
    sp = jnp.sum(p)
    sp2 = jnp.sum(p2)
    sp3 = jnp.sum(p2 * p)
    sp4 = jnp.sum(p2 * p2)
    sb = jnp.sum(bce)
    lane = jax.lax.broadcasted_iota(jnp.int32, (1, 1, 128), 2)
    row = (jnp.where(lane == 0, sp, 0.0) + jnp.where(lane == 1, sp2, 0.0)
           + jnp.where(lane == 2, sp3, 0.0) + jnp.where(lane == 3, sp4, 0.0)
           + jnp.where(lane == 4, sb, 0.0))
    part_ref[...] = row


def _head(sq_row, acc_row, a_tabT, b_tabT, vrunT, wc_col, bc, *, n_labels,
          tlq, n_nodes, kshift, sw):
    n_tiles = n_labels // tlq
    h = a_tabT.shape[0]
    na = a_tabT.shape[1]
    ve = vrunT.shape[1]
    return pl.pallas_call(
        functools.partial(_head_kernel, n_nodes=n_nodes, kshift=kshift,
                          sw=sw),
        out_shape=(jax.ShapeDtypeStruct((1, n_labels), jnp.float32),
                   jax.ShapeDtypeStruct((n_tiles, 1, 128), jnp.float32)),
        grid=(n_tiles,),
        in_specs=[
            pl.BlockSpec((1, tlq), lambda i: (0, i)),
            pl.BlockSpec((1, tlq), lambda i: (0, i)),
            pl.BlockSpec((h, na), lambda i: (0, 0)),
            pl.BlockSpec((h, n_nodes), lambda i: (0, 0)),
            pl.BlockSpec((h, ve), lambda i: (0, 0)),
            pl.BlockSpec((h, 1), lambda i: (0, 0)),
            pl.BlockSpec((1, 1), lambda i: (0, 0)),
        ],
        out_specs=(pl.BlockSpec((1, tlq), lambda i: (0, i)),
                   pl.BlockSpec((1, 1, 128), lambda i: (i, 0, 0))),
        compiler_params=pltpu.CompilerParams(dimension_semantics=("parallel",)),
    )(sq_row, acc_row, a_tabT, b_tabT, vrunT, wc_col, bc)


def _loss_kernel(part_ref, loss_ref, *, n, reg):
    v = jnp.sum(part_ref[...], axis=0)
    sp, sp2, sp3, sp4, sbce = (v[:, 0:1], v[:, 1:2], v[:, 2:3], v[:, 3:4],
                               v[:, 4:5])
    nf = jnp.float32(n)
    mu = sp / nf
    mu2 = mu * mu
    m4 = (sp4 / nf - 4.0 * mu * (sp3 / nf)
          + 6.0 * mu2 * (sp2 / nf) - 3.0 * mu2 * mu2)
    m4 = jnp.maximum(m4, 0.0)
    loss_ref[...] = sbce / nf - jnp.sqrt(jnp.sqrt(m4)) * reg


def _finalize(part, *, n, reg):
    n_tiles = part.shape[0]
    return pl.pallas_call(
        functools.partial(_loss_kernel, n=n, reg=reg),
        out_shape=jax.ShapeDtypeStruct((1, 1), jnp.float32),
        grid=(1,),
        in_specs=[pl.BlockSpec((n_tiles, 1, 128), lambda i: (0, 0, 0))],
        out_specs=pl.BlockSpec((1, 1), lambda i: (0, 0)),
    )(part)


def kernel(x, edge_index, edge_label_index, edge_type,
           convN1_w, convN1_b, convE1_w, convE1_b,
           convN2_w, convN2_b, convE2_w, convE2_b, cls_w, cls_b):
    x = x.astype(jnp.float32)
    n_nodes, f_in = x.shape
    n_edges = edge_index.shape[1]
    n_labels = edge_label_index.shape[1]
    h = _HIDDEN

    src, dst = edge_index[0], edge_index[1]
    lsrc, ldst = edge_label_index[0], edge_label_index[1]

    ea0 = (x[dst] - x[src]).astype(jnp.float32)

    n_pad = _round_up(n_nodes, 8)
    tn = _pick_tile(n_pad, 512)
    e_pad = _round_up(n_edges, 128)
    te = _pick_tile(e_pad, 1024)

    dst_row = jnp.full((1, e_pad), -1, jnp.int32).at[0, :n_edges].set(
        dst.astype(jnp.int32))
    ea0p = _pad_rows(ea0, e_pad)

    we1 = convN1_w[f_in:2 * f_in]
    X1 = _agg1(dst_row, ea0p, we1, convN1_b,
               n_pad=n_pad, tn=tn, e_pad=e_pad, te=te)[:n_nodes]

    xs1 = _pad_rows(X1[src], e_pad)
    xd1 = _pad_rows(X1[dst], e_pad)

    w1, w2, w3 = convE1_w[:h], convE1_w[h:2 * h], convE1_w[2 * h:2 * h + f_in]
    wx2, we2 = convN2_w[:h], convN2_w[h:2 * h]
    w1h, w2h, w3h = convE2_w[:h], convE2_w[h:2 * h], convE2_w[2 * h:3 * h]
    te_b = _pick_tile(e_pad, 2048)
    M2, P = _edge_pass(xs1, xd1, ea0p, w1, w2, w3, convE1_b,
                       wx2, we2, convN2_b, w3h, e_pad=e_pad, te=te_b)

    a_tab, b_tab = _agg2(dst_row, M2, w1h, w2h, convE2_b,
                         n_pad=n_pad, tn=tn, e_pad=e_pad, te=te)
    a_tab, b_tab = a_tab[:n_nodes], b_tab[:n_nodes]

    kshift = max(int(n_nodes - 1).bit_length(), 1)
    kmul = 1 << kshift
    key = src.astype(jnp.int32) * kmul + dst.astype(jnp.int32)
    order = jnp.argsort(key)
    skey = key[order]
    ps = P[:n_edges][order]
    csum = jnp.concatenate(
        [jnp.zeros((1, h), jnp.float32), jnp.cumsum(ps, axis=0)], axis=0)
    is_start = jnp.concatenate(
        [jnp.ones((1,), jnp.bool_), skey[1:] != skey[:-1]])
    eidx = jnp.arange(n_edges, dtype=jnp.int32)
    run_start = jax.lax.cummax(jnp.where(is_start, eidx, 0))
    vrun = jnp.concatenate(
        [jnp.zeros((1, h), jnp.float32), csum[1:] - csum[run_start]], axis=0)

    qkey = lsrc.astype(jnp.int32) * kmul + ldst.astype(jnp.int32)
    key2 = (qkey << 1) | edge_type.astype(jnp.int32)
    liota = jnp.arange(n_labels, dtype=jnp.int32)
    sq, sidx = jax.lax.sort((key2, liota), num_keys=1)
    sqk = sq >> 1
    pl_pos = jnp.searchsorted(sqk, skey, side="left")
    pr_pos = jnp.searchsorted(sqk, skey, side="right")
    packed = (jnp.zeros((n_labels + 1,), jnp.int32)
              .at[pl_pos].add(65537)
              .at[pr_pos].add(-65536))
    acc = jnp.cumsum(packed)[:n_labels]

    a_tabT = _pad_rows(a_tab, _round_up(n_pad, 128)).T
    b_tabT = _pad_rows(b_tab, n_pad).T.astype(jnp.bfloat16)
    ve = _round_up(n_edges + 1, 128)
    vrunT = jnp.zeros((h, ve), jnp.float32).at[:, :n_edges + 1].set(vrun.T)
    tlq = _pick_tile(n_labels, 1024)
    pred_row, part = _head(
        sq.reshape(1, n_labels), acc.reshape(1, n_labels), a_tabT, b_tabT,
        vrunT, cls_w.reshape(h, 1), cls_b.reshape(1, 1),
        n_labels=n_labels, tlq=tlq, n_nodes=n_pad, kshift=kshift,
        sw=float(_SIMILAR_WEIGHT))

    loss = _finalize(part, n=n_labels, reg=float(_REGULARIZER))
    edge_pred = jnp.zeros((n_labels,), jnp.float32).at[sidx].set(pred_row[0])
    return edge_pred, loss[0, 0]


# speedup vs baseline: 37.5997x; 1.0021x over previous
import functools

import jax
import jax.numpy as jnp
from jax.experimental import pallas as pl
from jax.experimental.pallas import tpu as pltpu

_HIDDEN = 32
_SIMILAR_WEIGHT = 1.0
_REGULARIZER = 0.1


def _round_up(n, m):
    return ((n + m - 1) // m) * m


def _pick_tile(padded, want):
    return want if padded % want == 0 else padded


def _pad_rows(a, rows):
    pad = rows - a.shape[0]
    if pad == 0:
        return a
    return jnp.pad(a, ((0, pad), (0, 0)))


def _agg1_kernel(dst_ref, ea_ref, we_ref, b_ref, out_ref, acc_ref, deg_ref,
                 *, tn):
    i = pl.program_id(0)
    k = pl.program_id(1)

    @pl.when(k == 0)
    def _():
        acc_ref[...] = jnp.zeros_like(acc_ref)
        deg_ref[...] = jnp.zeros_like(deg_ref)

    msg = jnp.dot(ea_ref[...], we_ref[...], preferred_element_type=jnp.float32)
    msg = jnp.maximum(msg + b_ref[...], 0.0)
    te = msg.shape[0]
    ids = i * tn + jax.lax.broadcasted_iota(jnp.int32, (tn, te), 0)
    onehot = (dst_ref[...] == ids).astype(jnp.float32)
    acc_ref[...] += jnp.dot(onehot, msg, preferred_element_type=jnp.float32)
    deg_ref[...] += jnp.sum(onehot, axis=1, keepdims=True)

    @pl.when(k == pl.num_programs(1) - 1)
    def _():
        out_ref[...] = acc_ref[...] / jnp.maximum(deg_ref[...], 1.0)


def _agg1(dst_row, ea, we, b, *, n_pad, tn, e_pad, te):
    h = we.shape[1]
    fe = ea.shape[1]
    return pl.pallas_call(
        functools.partial(_agg1_kernel, tn=tn),
        out_shape=jax.ShapeDtypeStruct((n_pad, h), jnp.float32),
        grid=(n_pad // tn, e_pad // te),
        in_specs=[
            pl.BlockSpec((1, te), lambda i, k: (0, k)),
            pl.BlockSpec((te, fe), lambda i, k: (k, 0)),
            pl.BlockSpec((fe, h), lambda i, k: (0, 0)),
            pl.BlockSpec((1, h), lambda i, k: (0, 0)),
        ],
        out_specs=pl.BlockSpec((tn, h), lambda i, k: (i, 0)),
        scratch_shapes=[pltpu.VMEM((tn, h), jnp.float32),
                        pltpu.VMEM((tn, 1), jnp.float32)],
        compiler_params=pltpu.CompilerParams(
            dimension_semantics=("parallel", "arbitrary")),
    )(dst_row, ea, we, b.reshape(1, h))


def _edge_kernel(xs_ref, xd_ref, ea_ref, w1_ref, w2_ref, w3_ref, b_ref,
                 wx2_ref, we2_ref, b2_ref, w3h_ref, m2_ref, p_ref):
    ea1 = jnp.dot(xs_ref[...], w1_ref[...], preferred_element_type=jnp.float32)
    ea1 = ea1 + jnp.dot(xd_ref[...], w2_ref[...],
                        preferred_element_type=jnp.float32)
    ea1 = ea1 + jnp.dot(ea_ref[...], w3_ref[...],
                        preferred_element_type=jnp.float32)
    ea1 = jnp.maximum(ea1 + b_ref[...], 0.0)
    m2 = jnp.dot(xs_ref[...], wx2_ref[...], preferred_element_type=jnp.float32)
    m2 = m2 + jnp.dot(ea1, we2_ref[...], preferred_element_type=jnp.float32)
    m2_ref[...] = jnp.maximum(m2 + b2_ref[...], 0.0)
    p_ref[...] = jnp.dot(ea1, w3h_ref[...], preferred_element_type=jnp.float32)


def _edge_pass(xs, xd, ea, w1, w2, w3, b, wx2, we2, b2, w3h, *, e_pad, te):
    h = w1.shape[1]
    fn = xs.shape[1]
    fe = ea.shape[1]
    return pl.pallas_call(
        _edge_kernel,
        out_shape=(jax.ShapeDtypeStruct((e_pad, h), jnp.float32),
                   jax.ShapeDtypeStruct((e_pad, h), jnp.float32)),
        grid=(e_pad // te,),
        in_specs=[
            pl.BlockSpec((te, fn), lambda i: (i, 0)),
            pl.BlockSpec((te, fn), lambda i: (i, 0)),
            pl.BlockSpec((te, fe), lambda i: (i, 0)),
            pl.BlockSpec((fn, h), lambda i: (0, 0)),
            pl.BlockSpec((fn, h), lambda i: (0, 0)),
            pl.BlockSpec((fe, h), lambda i: (0, 0)),
            pl.BlockSpec((1, h), lambda i: (0, 0)),
            pl.BlockSpec((h, h), lambda i: (0, 0)),
            pl.BlockSpec((h, h), lambda i: (0, 0)),
            pl.BlockSpec((1, h), lambda i: (0, 0)),
            pl.BlockSpec((h, h), lambda i: (0, 0)),
        ],
        out_specs=(pl.BlockSpec((te, h), lambda i: (i, 0)),
                   pl.BlockSpec((te, h), lambda i: (i, 0))),
        compiler_params=pltpu.CompilerParams(dimension_semantics=("parallel",)),
    )(xs, xd, ea, w1, w2, w3, b.reshape(1, h), wx2, we2, b2.reshape(1, h), w3h)


def _agg2_kernel(dst_ref, m2_ref, w1h_ref, w2h_ref, bh_ref,
                 a_ref, b_out_ref, acc_ref, deg_ref, *, tn):
    i = pl.program_id(0)
    k = pl.program_id(1)

    @pl.when(k == 0)
    def _():
        acc_ref[...] = jnp.zeros_like(acc_ref)
        deg_ref[...] = jnp.zeros_like(deg_ref)

    m2 = m2_ref[...]
    te = m2.shape[0]
    ids = i * tn + jax.lax.broadcasted_iota(jnp.int32, (tn, te), 0)
    onehot = (dst_ref[...] == ids).astype(jnp.float32)
    acc_ref[...] += jnp.dot(onehot, m2, preferred_element_type=jnp.float32)
    deg_ref[...] += jnp.sum(onehot, axis=1, keepdims=True)

    @pl.when(k == pl.num_programs(1) - 1)
    def _():
        x2 = acc_ref[...] / jnp.maximum(deg_ref[...], 1.0)
        a_ref[...] = jnp.dot(x2, w1h_ref[...],
                             preferred_element_type=jnp.float32) + bh_ref[...]
        b_out_ref[...] = jnp.dot(x2, w2h_ref[...],
                                 preferred_element_type=jnp.float32)


def _agg2(dst_row, m2, w1h, w2h, bh, *, n_pad, tn, e_pad, te):
    h = w1h.shape[1]
    return pl.pallas_call(
        functools.partial(_agg2_kernel, tn=tn),
        out_shape=(jax.ShapeDtypeStruct((n_pad, h), jnp.float32),
                   jax.ShapeDtypeStruct((n_pad, h), jnp.float32)),
        grid=(n_pad // tn, e_pad // te),
        in_specs=[
            pl.BlockSpec((1, te), lambda i, k: (0, k)),
            pl.BlockSpec((te, h), lambda i, k: (k, 0)),
            pl.BlockSpec((h, h), lambda i, k: (0, 0)),
            pl.BlockSpec((h, h), lambda i, k: (0, 0)),
            pl.BlockSpec((1, h), lambda i, k: (0, 0)),
        ],
        out_specs=(pl.BlockSpec((tn, h), lambda i, k: (i, 0)),
                   pl.BlockSpec((tn, h), lambda i, k: (i, 0))),
        scratch_shapes=[pltpu.VMEM((tn, h), jnp.float32),
                        pltpu.VMEM((tn, 1), jnp.float32)],
        compiler_params=pltpu.CompilerParams(
            dimension_semantics=("parallel", "arbitrary")),
    )(dst_row, m2, w1h, w2h, bh.reshape(1, h))


def _head_kernel(sq_ref, acc_ref, at_ref, bt_ref, vt_ref, wc_ref, bc_ref,
                 pred_ref, part_ref, *, n_nodes, kshift, sw):
    sqv = sq_ref[...]
    lsrc = sqv >> (kshift + 1)
    ldst = (sqv >> 1) & ((1 << kshift) - 1)
    tlq = sqv.shape[1]
    subl = jax.lax.broadcasted_iota(jnp.int32, (n_nodes, tlq), 0)
    oh_b = (subl == ldst).astype(jnp.bfloat16)
    gt = jnp.dot(bt_ref[...], oh_b, preferred_element_type=jnp.float32)
    h = gt.shape[0]
    lane128 = jax.lax.broadcasted_iota(jnp.int32, (1, 128), 1)

    a_lo = jnp.min(lsrc)
    a_hi = jnp.max(lsrc)

    def body_a(a, ga):
        cbase = pl.multiple_of((a >> 7) * 128, 128)
        chunk = at_ref[:, pl.ds(cbase, 128)]
        lmask = (lane128 == (a & 127)).astype(jnp.float32)
        col = jnp.sum(chunk * lmask, axis=1, keepdims=True)
        return ga + col * (lsrc == a).astype(jnp.float32)

    gt = gt + jax.lax.fori_loop(a_lo, a_hi + 1, body_a,
                                jnp.zeros((h, tlq), jnp.float32))

    accv = acc_ref[...]
    hiv = accv & 0xFFFF
    pres = (accv >> 16) > 0
    lo_s = jnp.min(jnp.where(pres, hiv, jnp.int32(1 << 30)))
    hi_s = jnp.max(jnp.where(pres, hiv, jnp.int32(-1)))

    def body(a, ct):
        cbase = pl.multiple_of((a >> 7) * 128, 128)
        chunk = vt_ref[:, pl.ds(cbase, 128)]
        lmask = (lane128 == (a & 127)).astype(jnp.float32)
        col = jnp.sum(chunk * lmask, axis=1, keepdims=True)
        qmask = (hiv == a).astype(jnp.float32)
        return ct + col * qmask

    ct = jax.lax.fori_loop(lo_s, hi_s + 1, body,
                           jnp.zeros((h, tlq), jnp.float32))
    gt = gt + ct * pres.astype(jnp.float32)
    hh = jnp.maximum(gt, 0.0)
    logits = jnp.sum(hh * wc_ref[...], axis=0, keepdims=True) + bc_ref[...]
    p = jax.nn.sigmoid(logits)
    pred_ref[...] = p

    y = (sqv & 1).astype(jnp.float32)
    w = jnp.where(p > 0.5, jnp.float32(sw), jnp.float32(1.0))
    log_p = jnp.maximum(jnp.log(p), -100.0)
    log_1mp = jnp.maximum(jnp.log(1.0 - p), -100.0)
    bce = w * -(y * log_p + (1.0 - y) * log_1mp)

    p2 = p * p
    sp = jnp.sum(p)
    sp2 = jnp.sum(p2)
    sp3 = jnp.sum(p2 * p)
    sp4 = jnp.sum(p2 * p2)
    sb = jnp.sum(bce)
    lane = jax.lax.broadcasted_iota(jnp.int32, (1, 1, 128), 2)
    row = (jnp.where(lane == 0, sp, 0.0) + jnp.where(lane == 1, sp2, 0.0)
           + jnp.where(lane == 2, sp3, 0.0) + jnp.where(lane == 3, sp4, 0.0)
           + jnp.where(lane == 4, sb, 0.0))
    part_ref[...] = row


def _head(sq_row, acc_row, a_tabT, b_tabT, vrunT, wc_col, bc, *, n_labels,
          tlq, n_nodes, kshift, sw):
    n_tiles = n_labels // tlq
    h = a_tabT.shape[0]
    na = a_tabT.shape[1]
    ve = vrunT.shape[1]
    return pl.pallas_call(
        functools.partial(_head_kernel, n_nodes=n_nodes, kshift=kshift,
                          sw=sw),
        out_shape=(jax.ShapeDtypeStruct((1, n_labels), jnp.float32),
                   jax.ShapeDtypeStruct((n_tiles, 1, 128), jnp.float32)),
        grid=(n_tiles,),
        in_specs=[
            pl.BlockSpec((1, tlq), lambda i: (0, i)),
            pl.BlockSpec((1, tlq), lambda i: (0, i)),
            pl.BlockSpec((h, na), lambda i: (0, 0)),
            pl.BlockSpec((h, n_nodes), lambda i: (0, 0)),
            pl.BlockSpec((h, ve), lambda i: (0, 0)),
            pl.BlockSpec((h, 1), lambda i: (0, 0)),
            pl.BlockSpec((1, 1), lambda i: (0, 0)),
        ],
        out_specs=(pl.BlockSpec((1, tlq), lambda i: (0, i)),
                   pl.BlockSpec((1, 1, 128), lambda i: (i, 0, 0))),
        compiler_params=pltpu.CompilerParams(dimension_semantics=("parallel",)),
    )(sq_row, acc_row, a_tabT, b_tabT, vrunT, wc_col, bc)


def _loss_kernel(part_ref, loss_ref, *, n, reg):
    v = jnp.sum(part_ref[...], axis=0)
    sp, sp2, sp3, sp4, sbce = (v[:, 0:1], v[:, 1:2], v[:, 2:3], v[:, 3:4],
                               v[:, 4:5])
    nf = jnp.float32(n)
    mu = sp / nf
    mu2 = mu * mu
    m4 = (sp4 / nf - 4.0 * mu * (sp3 / nf)
          + 6.0 * mu2 * (sp2 / nf) - 3.0 * mu2 * mu2)
    m4 = jnp.maximum(m4, 0.0)
    loss_ref[...] = sbce / nf - jnp.sqrt(jnp.sqrt(m4)) * reg


def _finalize(part, *, n, reg):
    n_tiles = part.shape[0]
    return pl.pallas_call(
        functools.partial(_loss_kernel, n=n, reg=reg),
        out_shape=jax.ShapeDtypeStruct((1, 1), jnp.float32),
        grid=(1,),
        in_specs=[pl.BlockSpec((n_tiles, 1, 128), lambda i: (0, 0, 0))],
        out_specs=pl.BlockSpec((1, 1), lambda i: (0, 0)),
    )(part)


def kernel(x, edge_index, edge_label_index, edge_type,
           convN1_w, convN1_b, convE1_w, convE1_b,
           convN2_w, convN2_b, convE2_w, convE2_b, cls_w, cls_b):
    x = x.astype(jnp.float32)
    n_nodes, f_in = x.shape
    n_edges = edge_index.shape[1]
    n_labels = edge_label_index.shape[1]
    h = _HIDDEN

    src, dst = edge_index[0], edge_index[1]
    lsrc, ldst = edge_label_index[0], edge_label_index[1]

    ea0 = (x[dst] - x[src]).astype(jnp.float32)

    n_pad = _round_up(n_nodes, 8)
    tn = _pick_tile(n_pad, 512)
    e_pad = _round_up(n_edges, 128)
    te = _pick_tile(e_pad, 1024)

    dst_row = jnp.full((1, e_pad), -1, jnp.int32).at[0, :n_edges].set(
        dst.astype(jnp.int32))
    ea0p = _pad_rows(ea0, e_pad)

    we1 = convN1_w[f_in:2 * f_in]
    X1 = _agg1(dst_row, ea0p, we1, convN1_b,
               n_pad=n_pad, tn=tn, e_pad=e_pad, te=te)[:n_nodes]

    xs1 = _pad_rows(X1[src], e_pad)
    xd1 = _pad_rows(X1[dst], e_pad)

    w1, w2, w3 = convE1_w[:h], convE1_w[h:2 * h], convE1_w[2 * h:2 * h + f_in]
    wx2, we2 = convN2_w[:h], convN2_w[h:2 * h]
    w1h, w2h, w3h = convE2_w[:h], convE2_w[h:2 * h], convE2_w[2 * h:3 * h]
    te_b = _pick_tile(e_pad, 2048)
    M2, P = _edge_pass(xs1, xd1, ea0p, w1, w2, w3, convE1_b,
                       wx2, we2, convN2_b, w3h, e_pad=e_pad, te=te_b)

    a_tab, b_tab = _agg2(dst_row, M2, w1h, w2h, convE2_b,
                         n_pad=n_pad, tn=tn, e_pad=e_pad, te=te)
    a_tab, b_tab = a_tab[:n_nodes], b_tab[:n_nodes]

    kshift = max(int(n_nodes - 1).bit_length(), 1)
    kmul = 1 << kshift
    key = src.astype(jnp.int32) * kmul + dst.astype(jnp.int32)
    order = jnp.argsort(key)
    skey = key[order]
    ps = P[:n_edges][order]
    csum = jnp.concatenate(
        [jnp.zeros((1, h), jnp.float32), jnp.cumsum(ps, axis=0)], axis=0)
    is_start = jnp.concatenate(
        [jnp.ones((1,), jnp.bool_), skey[1:] != skey[:-1]])
    eidx = jnp.arange(n_edges, dtype=jnp.int32)
    run_start = jax.lax.cummax(jnp.where(is_start, eidx, 0))
    vrun = jnp.concatenate(
        [jnp.zeros((1, h), jnp.float32), csum[1:] - csum[run_start]], axis=0)

    qkey = lsrc.astype(jnp.int32) * kmul + ldst.astype(jnp.int32)
    key2 = (qkey << 1) | edge_type.astype(jnp.int32)
    liota = jnp.arange(n_labels, dtype=jnp.int32)
    sq, sidx = jax.lax.sort((key2, liota), num_keys=1)
    sqk = sq >> 1
    pl_pos = jnp.searchsorted(sqk, skey, side="left")
    pr_pos = jnp.searchsorted(sqk, skey, side="right")
    sc_idx = jnp.concatenate([pl_pos, pr_pos])
    sc_val = jnp.concatenate([
        jnp.full((n_edges,), 65537, jnp.int32),
        jnp.full((n_edges,), -65536, jnp.int32)])
    packed = jnp.zeros((n_labels + 1,), jnp.int32).at[sc_idx].add(sc_val)
    acc = jnp.cumsum(packed)[:n_labels]

    a_tabT = _pad_rows(a_tab, _round_up(n_pad, 128)).T
    b_tabT = _pad_rows(b_tab, n_pad).T.astype(jnp.bfloat16)
    ve = _round_up(n_edges + 1, 128)
    vrunT = jnp.zeros((h, ve), jnp.float32).at[:, :n_edges + 1].set(vrun.T)
    tlq = _pick_tile(n_labels, 1024)
    pred_row, part = _head(
        sq.reshape(1, n_labels), acc.reshape(1, n_labels), a_tabT, b_tabT,
        vrunT, cls_w.reshape(h, 1), cls_b.reshape(1, 1),
        n_labels=n_labels, tlq=tlq, n_nodes=n_pad, kshift=kshift,
        sw=float(_SIMILAR_WEIGHT))

    loss = _finalize(part, n=n_labels, reg=float(_REGULARIZER))
    edge_pred = jnp.zeros((n_labels,), jnp.float32).at[sidx].set(pred_row[0])
    return edge_pred, loss[0, 0]


# speedup vs baseline: 37.6277x; 1.0007x over previous
import functools

import jax
import jax.numpy as jnp
from jax.experimental import pallas as pl
from jax.experimental.pallas import tpu as pltpu

_HIDDEN = 32
_SIMILAR_WEIGHT = 1.0
_REGULARIZER = 0.1


def _round_up(n, m):
    return ((n + m - 1) // m) * m


def _pick_tile(padded, want):
    return want if padded % want == 0 else padded


def _pad_rows(a, rows):
    pad = rows - a.shape[0]
    if pad == 0:
        return a
    return jnp.pad(a, ((0, pad), (0, 0)))


def _agg1_kernel(dst_ref, ea_ref, we_ref, b_ref, out_ref, acc_ref, deg_ref,
                 *, tn):
    i = pl.program_id(0)
    k = pl.program_id(1)

    @pl.when(k == 0)
    def _():
        acc_ref[...] = jnp.zeros_like(acc_ref)
        deg_ref[...] = jnp.zeros_like(deg_ref)

    msg = jnp.dot(ea_ref[...], we_ref[...], preferred_element_type=jnp.float32)
    msg = jnp.maximum(msg + b_ref[...], 0.0)
    te = msg.shape[0]
    ids = i * tn + jax.lax.broadcasted_iota(jnp.int32, (tn, te), 0)
    onehot = (dst_ref[...] == ids).astype(jnp.float32)
    acc_ref[...] += jnp.dot(onehot, msg, preferred_element_type=jnp.float32)
    deg_ref[...] += jnp.sum(onehot, axis=1, keepdims=True)

    @pl.when(k == pl.num_programs(1) - 1)
    def _():
        out_ref[...] = acc_ref[...] / jnp.maximum(deg_ref[...], 1.0)


def _agg1(dst_row, ea, we, b, *, n_pad, tn, e_pad, te):
    h = we.shape[1]
    fe = ea.shape[1]
    return pl.pallas_call(
        functools.partial(_agg1_kernel, tn=tn),
        out_shape=jax.ShapeDtypeStruct((n_pad, h), jnp.float32),
        grid=(n_pad // tn, e_pad // te),
        in_specs=[
            pl.BlockSpec((1, te), lambda i, k: (0, k)),
            pl.BlockSpec((te, fe), lambda i, k: (k, 0)),
            pl.BlockSpec((fe, h), lambda i, k: (0, 0)),
            pl.BlockSpec((1, h), lambda i, k: (0, 0)),
        ],
        out_specs=pl.BlockSpec((tn, h), lambda i, k: (i, 0)),
        scratch_shapes=[pltpu.VMEM((tn, h), jnp.float32),
                        pltpu.VMEM((tn, 1), jnp.float32)],
        compiler_params=pltpu.CompilerParams(
            dimension_semantics=("parallel", "arbitrary")),
    )(dst_row, ea, we, b.reshape(1, h))


def _edge_kernel(xs_ref, xd_ref, ea_ref, w1_ref, w2_ref, w3_ref, b_ref,
                 wx2_ref, we2_ref, b2_ref, w3h_ref, m2_ref, p_ref):
    ea1 = jnp.dot(xs_ref[...], w1_ref[...], preferred_element_type=jnp.float32)
    ea1 = ea1 + jnp.dot(xd_ref[...], w2_ref[...],
                        preferred_element_type=jnp.float32)
    ea1 = ea1 + jnp.dot(ea_ref[...], w3_ref[...],
                        preferred_element_type=jnp.float32)
    ea1 = jnp.maximum(ea1 + b_ref[...], 0.0)
    m2 = jnp.dot(xs_ref[...], wx2_ref[...], preferred_element_type=jnp.float32)
    m2 = m2 + jnp.dot(ea1, we2_ref[...], preferred_element_type=jnp.float32)
    m2_ref[...] = jnp.maximum(m2 + b2_ref[...], 0.0)
    p_ref[...] = jnp.dot(ea1, w3h_ref[...], preferred_element_type=jnp.float32)


def _edge_pass(xs, xd, ea, w1, w2, w3, b, wx2, we2, b2, w3h, *, e_pad, te):
    h = w1.shape[1]
    fn = xs.shape[1]
    fe = ea.shape[1]
    return pl.pallas_call(
        _edge_kernel,
        out_shape=(jax.ShapeDtypeStruct((e_pad, h), jnp.float32),
                   jax.ShapeDtypeStruct((e_pad, h), jnp.float32)),
        grid=(e_pad // te,),
        in_specs=[
            pl.BlockSpec((te, fn), lambda i: (i, 0)),
            pl.BlockSpec((te, fn), lambda i: (i, 0)),
            pl.BlockSpec((te, fe), lambda i: (i, 0)),
            pl.BlockSpec((fn, h), lambda i: (0, 0)),
            pl.BlockSpec((fn, h), lambda i: (0, 0)),
            pl.BlockSpec((fe, h), lambda i: (0, 0)),
            pl.BlockSpec((1, h), lambda i: (0, 0)),
            pl.BlockSpec((h, h), lambda i: (0, 0)),
            pl.BlockSpec((h, h), lambda i: (0, 0)),
            pl.BlockSpec((1, h), lambda i: (0, 0)),
            pl.BlockSpec((h, h), lambda i: (0, 0)),
        ],
        out_specs=(pl.BlockSpec((te, h), lambda i: (i, 0)),
                   pl.BlockSpec((te, h), lambda i: (i, 0))),
        compiler_params=pltpu.CompilerParams(dimension_semantics=("parallel",)),
    )(xs, xd, ea, w1, w2, w3, b.reshape(1, h), wx2, we2, b2.reshape(1, h), w3h)


def _agg2_kernel(dst_ref, m2_ref, w1h_ref, w2h_ref, bh_ref,
                 a_ref, b_out_ref, acc_ref, deg_ref, *, tn):
    i = pl.program_id(0)
    k = pl.program_id(1)

    @pl.when(k == 0)
    def _():
        acc_ref[...] = jnp.zeros_like(acc_ref)
        deg_ref[...] = jnp.zeros_like(deg_ref)

    m2 = m2_ref[...]
    te = m2.shape[0]
    ids = i * tn + jax.lax.broadcasted_iota(jnp.int32, (tn, te), 0)
    onehot = (dst_ref[...] == ids).astype(jnp.float32)
    acc_ref[...] += jnp.dot(onehot, m2, preferred_element_type=jnp.float32)
    deg_ref[...] += jnp.sum(onehot, axis=1, keepdims=True)

    @pl.when(k == pl.num_programs(1) - 1)
    def _():
        x2 = acc_ref[...] / jnp.maximum(deg_ref[...], 1.0)
        a_ref[...] = jnp.dot(x2, w1h_ref[...],
                             preferred_element_type=jnp.float32) + bh_ref[...]
        b_out_ref[...] = jnp.dot(x2, w2h_ref[...],
                                 preferred_element_type=jnp.float32)


def _agg2(dst_row, m2, w1h, w2h, bh, *, n_pad, tn, e_pad, te):
    h = w1h.shape[1]
    return pl.pallas_call(
        functools.partial(_agg2_kernel, tn=tn),
        out_shape=(jax.ShapeDtypeStruct((n_pad, h), jnp.float32),
                   jax.ShapeDtypeStruct((n_pad, h), jnp.float32)),
        grid=(n_pad // tn, e_pad // te),
        in_specs=[
            pl.BlockSpec((1, te), lambda i, k: (0, k)),
            pl.BlockSpec((te, h), lambda i, k: (k, 0)),
            pl.BlockSpec((h, h), lambda i, k: (0, 0)),
            pl.BlockSpec((h, h), lambda i, k: (0, 0)),
            pl.BlockSpec((1, h), lambda i, k: (0, 0)),
        ],
        out_specs=(pl.BlockSpec((tn, h), lambda i, k: (i, 0)),
                   pl.BlockSpec((tn, h), lambda i, k: (i, 0))),
        scratch_shapes=[pltpu.VMEM((tn, h), jnp.float32),
                        pltpu.VMEM((tn, 1), jnp.float32)],
        compiler_params=pltpu.CompilerParams(
            dimension_semantics=("parallel", "arbitrary")),
    )(dst_row, m2, w1h, w2h, bh.reshape(1, h))


def _head_kernel(sq_ref, acc_ref, at_ref, bt_ref, vt_ref, wc_ref, bc_ref,
                 pred_ref, part_ref, *, n_nodes, kshift, sw):
    sqv = sq_ref[...]
    lsrc = sqv >> (kshift + 1)
    ldst = (sqv >> 1) & ((1 << kshift) - 1)
    tlq = sqv.shape[1]
    subl = jax.lax.broadcasted_iota(jnp.int32, (n_nodes, tlq), 0)
    oh_b = (subl == ldst).astype(jnp.bfloat16)
    gt = jnp.dot(bt_ref[...], oh_b, preferred_element_type=jnp.float32)
    h = gt.shape[0]
    lane128 = jax.lax.broadcasted_iota(jnp.int32, (1, 128), 1)

    a_lo = jnp.min(lsrc)
    a_hi = jnp.max(lsrc)

    def body_a(a, ga):
        cbase = pl.multiple_of((a >> 7) * 128, 128)
        chunk = at_ref[:, pl.ds(cbase, 128)]
        lmask = (lane128 == (a & 127)).astype(jnp.float32)
        col = jnp.sum(chunk * lmask, axis=1, keepdims=True)
        return ga + col * (lsrc == a).astype(jnp.float32)

    gt = gt + jax.lax.fori_loop(a_lo, a_hi + 1, body_a,
                                jnp.zeros((h, tlq), jnp.float32))

    accv = acc_ref[...]
    hiv = accv & 0xFFFF
    pres = (accv >> 16) > 0
    lo_s = jnp.min(jnp.where(pres, hiv, jnp.int32(1 << 30)))
    hi_s = jnp.max(jnp.where(pres, hiv, jnp.int32(-1)))

    def body(a, ct):
        cbase = pl.multiple_of((a >> 7) * 128, 128)
        chunk = vt_ref[:, pl.ds(cbase, 128)]
        lmask = (lane128 == (a & 127)).astype(jnp.float32)
        col = jnp.sum(chunk * lmask, axis=1, keepdims=True)
        qmask = (hiv == a).astype(jnp.float32)
        return ct + col * qmask

    ct = jax.lax.fori_loop(lo_s, hi_s + 1, body,
                           jnp.zeros((h, tlq), jnp.float32))
    gt = gt + ct * pres.astype(jnp.float32)
    hh = jnp.maximum(gt, 0.0)
    logits = jnp.sum(hh * wc_ref[...], axis=0, keepdims=True) + bc_ref[...]
    p = jax.nn.sigmoid(logits)
    pred_ref[...] = p

    y = (sqv & 1).astype(jnp.float32)
    w = jnp.where(p > 0.5, jnp.float32(sw), jnp.float32(1.0))
    log_p = jnp.maximum(jnp.log(p), -100.0)
    log_1mp = jnp.maximum(jnp.log(1.0 - p), -100.0)
    bce = w * -(y * log_p + (1.0 - y) * log_1mp)

    p2 = p * p
    sp = jnp.sum(p)
    sp2 = jnp.sum(p2)
    sp3 = jnp.sum(p2 * p)
    sp4 = jnp.sum(p2 * p2)
    sb = jnp.sum(bce)
    lane = jax.lax.broadcasted_iota(jnp.int32, (1, 1, 128), 2)
    row = (jnp.where(lane == 0, sp, 0.0) + jnp.where(lane == 1, sp2, 0.0)
           + jnp.where(lane == 2, sp3, 0.0) + jnp.where(lane == 3, sp4, 0.0)
           + jnp.where(lane == 4, sb, 0.0))
    part_ref[...] = row


def _head(sq_row, acc_row, a_tabT, b_tabT, vrunT, wc_col, bc, *, n_labels,
          tlq, n_nodes, kshift, sw):
    n_tiles = n_labels // tlq
    h = a_tabT.shape[0]
    na = a_tabT.shape[1]
    ve = vrunT.shape[1]
    return pl.pallas_call(
        functools.partial(_head_kernel, n_nodes=n_nodes, kshift=kshift,
                          sw=sw),
        out_shape=(jax.ShapeDtypeStruct((1, n_labels), jnp.float32),
                   jax.ShapeDtypeStruct((n_tiles, 1, 128), jnp.float32)),
        grid=(n_tiles,),
        in_specs=[
            pl.BlockSpec((1, tlq), lambda i: (0, i)),
            pl.BlockSpec((1, tlq), lambda i: (0, i)),
            pl.BlockSpec((h, na), lambda i: (0, 0)),
            pl.BlockSpec((h, n_nodes), lambda i: (0, 0)),
            pl.BlockSpec((h, ve), lambda i: (0, 0)),
            pl.BlockSpec((h, 1), lambda i: (0, 0)),
            pl.BlockSpec((1, 1), lambda i: (0, 0)),
        ],
        out_specs=(pl.BlockSpec((1, tlq), lambda i: (0, i)),
                   pl.BlockSpec((1, 1, 128), lambda i: (i, 0, 0))),
        compiler_params=pltpu.CompilerParams(dimension_semantics=("parallel",)),
    )(sq_row, acc_row, a_tabT, b_tabT, vrunT, wc_col, bc)


def _loss_kernel(part_ref, loss_ref, *, n, reg):
    v = jnp.sum(part_ref[...], axis=0)
    sp, sp2, sp3, sp4, sbce = (v[:, 0:1], v[:, 1:2], v[:, 2:3], v[:, 3:4],
                               v[:, 4:5])
    nf = jnp.float32(n)
    mu = sp / nf
    mu2 = mu * mu
    m4 = (sp4 / nf - 4.0 * mu * (sp3 / nf)
          + 6.0 * mu2 * (sp2 / nf) - 3.0 * mu2 * mu2)
    m4 = jnp.maximum(m4, 0.0)
    loss_ref[...] = sbce / nf - jnp.sqrt(jnp.sqrt(m4)) * reg


def _finalize(part, *, n, reg):
    n_tiles = part.shape[0]
    return pl.pallas_call(
        functools.partial(_loss_kernel, n=n, reg=reg),
        out_shape=jax.ShapeDtypeStruct((1, 1), jnp.float32),
        grid=(1,),
        in_specs=[pl.BlockSpec((n_tiles, 1, 128), lambda i: (0, 0, 0))],
        out_specs=pl.BlockSpec((1, 1), lambda i: (0, 0)),
    )(part)


def kernel(x, edge_index, edge_label_index, edge_type,
           convN1_w, convN1_b, convE1_w, convE1_b,
           convN2_w, convN2_b, convE2_w, convE2_b, cls_w, cls_b):
    x = x.astype(jnp.float32)
    n_nodes, f_in = x.shape
    n_edges = edge_index.shape[1]
    n_labels = edge_label_index.shape[1]
    h = _HIDDEN

    src, dst = edge_index[0], edge_index[1]
    lsrc, ldst = edge_label_index[0], edge_label_index[1]

    ea0 = (x[dst] - x[src]).astype(jnp.float32)

    n_pad = _round_up(n_nodes, 8)
    tn = _pick_tile(n_pad, 512)
    e_pad = _round_up(n_edges, 128)
    te = _pick_tile(e_pad, 1024)

    dst_row = jnp.full((1, e_pad), -1, jnp.int32).at[0, :n_edges].set(
        dst.astype(jnp.int32))
    ea0p = _pad_rows(ea0, e_pad)

    we1 = convN1_w[f_in:2 * f_in]
    X1 = _agg1(dst_row, ea0p, we1, convN1_b,
               n_pad=n_pad, tn=tn, e_pad=e_pad, te=te)[:n_nodes]

    xs1 = _pad_rows(X1[src], e_pad)
    xd1 = _pad_rows(X1[dst], e_pad)

    w1, w2, w3 = convE1_w[:h], convE1_w[h:2 * h], convE1_w[2 * h:2 * h + f_in]
    wx2, we2 = convN2_w[:h], convN2_w[h:2 * h]
    w1h, w2h, w3h = convE2_w[:h], convE2_w[h:2 * h], convE2_w[2 * h:3 * h]
    te_b = _pick_tile(e_pad, 2048)
    M2, P = _edge_pass(xs1, xd1, ea0p, w1, w2, w3, convE1_b,
                       wx2, we2, convN2_b, w3h, e_pad=e_pad, te=te_b)

    a_tab, b_tab = _agg2(dst_row, M2, w1h, w2h, convE2_b,
                         n_pad=n_pad, tn=tn, e_pad=e_pad, te=te)
    a_tab, b_tab = a_tab[:n_nodes], b_tab[:n_nodes]

    kshift = max(int(n_nodes - 1).bit_length(), 1)
    kmul = 1 << kshift
    key = src.astype(jnp.int32) * kmul + dst.astype(jnp.int32)
    order = jnp.argsort(key)
    skey = key[order]
    ps = P[:n_edges][order]
    csum = jnp.concatenate(
        [jnp.zeros((1, h), jnp.float32), jnp.cumsum(ps, axis=0)], axis=0)
    is_start = jnp.concatenate(
        [jnp.ones((1,), jnp.bool_), skey[1:] != skey[:-1]])
    eidx = jnp.arange(n_edges, dtype=jnp.int32)
    run_start = jax.lax.cummax(jnp.where(is_start, eidx, 0))
    vrun = jnp.concatenate(
        [jnp.zeros((1, h), jnp.float32), csum[1:] - csum[run_start]], axis=0)

    qkey = lsrc.astype(jnp.int32) * kmul + ldst.astype(jnp.int32)
    key2 = (qkey << 1) | edge_type.astype(jnp.int32)
    liota = jnp.arange(n_labels, dtype=jnp.int32)
    sq, sidx = jax.lax.sort((key2, liota), num_keys=1)
    sqk = sq >> 1
    pl_pos = jnp.searchsorted(sqk, skey, side="left")
    pr_pos = jnp.searchsorted(sqk, skey, side="right")
    sc_idx = jnp.concatenate([pl_pos, pr_pos])
    sc_val = jnp.concatenate([
        jnp.full((n_edges,), 65537, jnp.int32),
        jnp.full((n_edges,), -65536, jnp.int32)])
    packed = jnp.zeros((n_labels + 1,), jnp.int32).at[sc_idx].add(sc_val)
    acc = jnp.cumsum(packed)[:n_labels]

    a_tabT = _pad_rows(a_tab, _round_up(n_pad, 128)).T
    b_tabT = _pad_rows(b_tab, n_pad).T.astype(jnp.bfloat16)
    ve = _round_up(n_edges + 1, 128)
    vrunT = jnp.zeros((h, ve), jnp.float32).at[:, :n_edges + 1].set(vrun.T)
    tlq = _pick_tile(n_labels, 2048)
    pred_row, part = _head(
        sq.reshape(1, n_labels), acc.reshape(1, n_labels), a_tabT, b_tabT,
        vrunT, cls_w.reshape(h, 1), cls_b.reshape(1, 1),
        n_labels=n_labels, tlq=tlq, n_nodes=n_pad, kshift=kshift,
        sw=float(_SIMILAR_WEIGHT))

    loss = _finalize(part, n=n_labels, reg=float(_REGULARIZER))
    edge_pred = jnp.zeros((n_labels,), jnp.float32).at[sidx].set(pred_row[0])
    return edge_pred, loss[0, 0]
